# Initial kernel scaffold; baseline (speedup 1.0000x reference)
import jax
import jax.numpy as jnp
from jax.experimental import pallas as pl


def kernel(x, edge_index, edge_weight, W1, We1, as1, ad1, ae1, b1, W2, We2, as2, ad2, ae2, b2):
    raise NotImplementedError("write your pallas kernel here")


# speedup vs baseline: 12.1268x; 12.1268x over previous
import functools

import jax
import jax.numpy as jnp
from jax import lax
from jax.experimental import pallas as pl
from jax.experimental.pallas import tpu as pltpu
from jax.experimental.pallas import tpu_sc as plsc

N = 10000
E = 320000
D = 128
DH = D // 2
NC = 2
NS = 16
EPT = E // NS
BE = 80
NBATCH = EPT // BE
RPT = 624
NTAIL = N - NS * RPT
BN = 512
GRID_N = (N + BN - 1) // BN


def _sc_edge_pass(hflat, aux, src, dst, ew, wa, z64, z16):
    mesh = plsc.VectorSubcoreMesh(core_axis_name="c", subcore_axis_name="s")

    @functools.partial(
        pl.kernel,
        out_type=(
            jax.ShapeDtypeStruct((NC * N, DH), jnp.float32),
            jax.ShapeDtypeStruct((N, 16), jnp.float32),
        ),
        mesh=mesh,
        compiler_params=pltpu.CompilerParams(needs_layout_passes=False,
                                             use_tc_tiling_on_sc=False),
        scratch_types=[
            pltpu.VMEM_SHARED((N, DH), jnp.float32),
            pltpu.VMEM_SHARED((N, 16), jnp.float32),
            pltpu.VMEM((N,), jnp.float32),
            pltpu.VMEM((N,), jnp.float32),
            pltpu.VMEM((2, D), jnp.float32),
            pltpu.VMEM((BE,), jnp.int32),
            pltpu.VMEM((BE,), jnp.int32),
            pltpu.VMEM((BE,), jnp.int32),
            pltpu.VMEM((BE,), jnp.float32),
            pltpu.VMEM((BE,), jnp.float32),
            pltpu.VMEM((BE, DH), jnp.float32),
            pltpu.VMEM((BE, 16), jnp.float32),
            pltpu.VMEM((16,), jnp.float32),
            pltpu.SemaphoreType.DMA,
        ],
    )
    def k(h_hbm, aux_hbm, src_hbm, dst_hbm, ew_hbm, wa_hbm, z64_hbm, z16_hbm,
          acc_out, den_out,
          acc_sh, den_sh, asrc_t, adst_t, wa_t, src_v, dst_v, gofs_v, ew_v,
          alpha_v, rows_v, tails_v, red_v, sem):
        c = lax.axis_index("c")
        s = lax.axis_index("s")
        pltpu.sync_copy(aux_hbm.at[0], asrc_t)
        pltpu.sync_copy(aux_hbm.at[1], adst_t)
        pltpu.sync_copy(wa_hbm, wa_t)
        pltpu.sync_copy(z16_hbm.at[pl.ds(0, BE)], tails_v)
        row0 = pl.multiple_of(s * RPT, 8)
        pltpu.sync_copy(z64_hbm, acc_sh.at[pl.ds(row0, RPT)])

        @pl.when(s == NS - 1)
        def _init_tail():
            pltpu.sync_copy(z64_hbm.at[pl.ds(0, NTAIL)],
                            acc_sh.at[pl.ds(NS * RPT, NTAIL)])

        @pl.when(c == 0)
        def _init_den():
            pltpu.sync_copy(z16_hbm, den_sh.at[pl.ds(row0, RPT)])

            @pl.when(s == NS - 1)
            def _init_den_tail():
                pltpu.sync_copy(z16_hbm.at[pl.ds(0, NTAIL)],
                                den_sh.at[pl.ds(NS * RPT, NTAIL)])

        lane16 = lax.iota(jnp.int32, 16)
        cev = jnp.zeros((16,), jnp.float32)
        for kk in range(D // 16):
            cev = cev + wa_t[0, pl.ds(kk * 16, 16)] * wa_t[1, pl.ds(kk * 16, 16)]
        for shift in (8, 4, 2, 1):
            red_v[...] = cev
            cev = cev + plsc.load_gather(red_v, [lane16 ^ shift])
        ce = cev
        col0 = jnp.zeros((16,), jnp.int32)
        col1 = col0 + 1
        col2 = col0 + 2
        ones16 = jnp.ones((16,), jnp.float32)
        gofs0 = c * N
        plsc.subcore_barrier()
        ebase0 = s * EPT

        def batch_body(b, carry):
            ebase = pl.multiple_of(ebase0 + b * BE, 8)
            pltpu.sync_copy(src_hbm.at[pl.ds(ebase, BE)], src_v)
            pltpu.sync_copy(dst_hbm.at[pl.ds(ebase, BE)], dst_v)
            pltpu.sync_copy(ew_hbm.at[pl.ds(ebase, BE)], ew_v)
            for g in range(BE // 16):
                si = src_v[pl.ds(g * 16, 16)]
                di = dst_v[pl.ds(g * 16, 16)]
                ew16 = ew_v[pl.ds(g * 16, 16)]
                a = (plsc.load_gather(asrc_t, [si])
                     + plsc.load_gather(adst_t, [di]) + ce * ew16)
                a = jnp.where(a > 0, a, a * jnp.float32(0.2))
                a = jnp.exp(a)
                alpha_v[pl.ds(g * 16, 16)] = a
                gofs_v[pl.ds(g * 16, 16)] = si + gofs0
                row16 = g * 16 + lane16
                plsc.store_scatter(tails_v, [row16, col0], a)
                plsc.store_scatter(tails_v, [row16, col1], ones16)
                plsc.store_scatter(tails_v, [row16, col2], ew16)
            gat = pltpu.async_copy(h_hbm.at[gofs_v], rows_v, sem)
            gat.wait()

            def row_body(r, carry2):
                spl = plsc.load_gather(alpha_v, [jnp.broadcast_to(r, (16,))])
                for cc in range(DH // 16):
                    v = rows_v[r, pl.ds(cc * 16, 16)]
                    rows_v[r, pl.ds(cc * 16, 16)] = v * spl
                return carry2

            lax.fori_loop(0, BE, row_body, 0, unroll=2)
            pltpu.sync_copy(rows_v, acc_sh.at[dst_v], add=True)

            @pl.when(c == 0)
            def _den_scatter():
                pltpu.sync_copy(tails_v, den_sh.at[dst_v], add=True)

            return carry

        lax.fori_loop(0, NBATCH, batch_body, 0)
        plsc.subcore_barrier()
        out0 = pl.multiple_of(c * N + row0, 8)
        pltpu.sync_copy(acc_sh.at[pl.ds(row0, RPT)],
                        acc_out.at[pl.ds(out0, RPT)])

        @pl.when(c == 0)
        def _out_den():
            pltpu.sync_copy(den_sh.at[pl.ds(row0, RPT)],
                            den_out.at[pl.ds(row0, RPT)])

        @pl.when(s == NS - 1)
        def _out_tail():
            t0 = pl.multiple_of(c * N + NS * RPT, 8)
            pltpu.sync_copy(acc_sh.at[pl.ds(NS * RPT, NTAIL)],
                            acc_out.at[pl.ds(t0, NTAIL)])

            @pl.when(c == 0)
            def _out_den_tail():
                pltpu.sync_copy(den_sh.at[pl.ds(NS * RPT, NTAIL)],
                                den_out.at[pl.ds(NS * RPT, NTAIL)])

    return k(hflat, aux, src, dst, ew, wa, z64, z16)


def _dot_t(a, b):
    return lax.dot_general(a, b, (((1,), (1,)), ((), ())),
                           preferred_element_type=jnp.float32)


def _split_cols(h, j):
    return jnp.where(j == 0, h[:, :DH], h[:, DH:])[None]


def _tc_embed(x, W, asv, adv):
    def body(x_ref, w_ref, as_ref, ad_ref, h_ref, aux_ref, hs_ref):
        h = _dot_t(x_ref[...], w_ref[...])
        h_ref[...] = h
        aux_ref[...] = jnp.zeros((8, BN), jnp.float32)
        aux_ref[0:1, :] = _dot_t(as_ref[...], h)
        aux_ref[1:2, :] = _dot_t(ad_ref[...], h)
        hs_ref[...] = _split_cols(h, pl.program_id(1))

    return pl.pallas_call(
        body,
        grid=(GRID_N, NC),
        in_specs=[
            pl.BlockSpec((BN, D), lambda i, j: (i, 0)),
            pl.BlockSpec((D, D), lambda i, j: (0, 0)),
            pl.BlockSpec((1, D), lambda i, j: (0, 0)),
            pl.BlockSpec((1, D), lambda i, j: (0, 0)),
        ],
        out_specs=[
            pl.BlockSpec((BN, D), lambda i, j: (i, 0)),
            pl.BlockSpec((8, BN), lambda i, j: (0, i)),
            pl.BlockSpec((1, BN, DH), lambda i, j: (j, i, 0)),
        ],
        out_shape=[
            jax.ShapeDtypeStruct((N, D), jnp.float32),
            jax.ShapeDtypeStruct((8, N), jnp.float32),
            jax.ShapeDtypeStruct((NC, N, DH), jnp.float32),
        ],
    )(x, W, asv, adv)


def _self_alpha(aux_ref, wa_ref, la):
    auxT = lax.dot_general(aux_ref[...], jnp.eye(8, dtype=jnp.float32),
                           (((0,), (0,)), ((), ())),
                           preferred_element_type=jnp.float32)
    wa = wa_ref[...]
    ce = jnp.sum(wa[0:1, :] * wa[1:2, :])
    logit = auxT[:, 0:1] + auxT[:, 1:2] + ce * la
    logit = jnp.where(logit > 0, logit, logit * jnp.float32(0.2))
    return jnp.exp(logit)


def _tc_combine1(acc, den, h1, aux1, wa1, b1, W2, as2v, ad2v):
    def body(alo_ref, ahi_ref, den_ref, h_ref, aux_ref, wa_ref, b_ref, w2_ref,
             as_ref, ad_ref, h2_ref, aux2_ref, la_ref, h2s_ref):
        dsum = den_ref[...]
        asum = dsum[:, 0:1]
        deg = dsum[:, 1:2]
        ews = dsum[:, 2:3]
        la = ews / jnp.maximum(deg, 1.0)
        aself = _self_alpha(aux_ref, wa_ref, la)
        dentot = asum + aself + jnp.float32(1e-16)
        accsum = jnp.concatenate([alo_ref[0], ahi_ref[0]], axis=-1)
        out1 = (accsum + aself * h_ref[...]) / dentot + b_ref[...]
        x2 = jnp.maximum(out1, 0.0)
        h2 = _dot_t(x2, w2_ref[...])
        h2_ref[...] = h2
        aux2_ref[...] = jnp.zeros((8, BN), jnp.float32)
        aux2_ref[0:1, :] = _dot_t(as_ref[...], h2)
        aux2_ref[1:2, :] = _dot_t(ad_ref[...], h2)
        la_ref[...] = la
        h2s_ref[...] = _split_cols(h2, pl.program_id(1))

    return pl.pallas_call(
        body,
        grid=(GRID_N, NC),
        in_specs=[
            pl.BlockSpec((1, BN, DH), lambda i, j: (0, i, 0)),
            pl.BlockSpec((1, BN, DH), lambda i, j: (1, i, 0)),
            pl.BlockSpec((BN, 16), lambda i, j: (i, 0)),
            pl.BlockSpec((BN, D), lambda i, j: (i, 0)),
            pl.BlockSpec((8, BN), lambda i, j: (0, i)),
            pl.BlockSpec((2, D), lambda i, j: (0, 0)),
            pl.BlockSpec((1, D), lambda i, j: (0, 0)),
            pl.BlockSpec((D, D), lambda i, j: (0, 0)),
            pl.BlockSpec((1, D), lambda i, j: (0, 0)),
            pl.BlockSpec((1, D), lambda i, j: (0, 0)),
        ],
        out_specs=[
            pl.BlockSpec((BN, D), lambda i, j: (i, 0)),
            pl.BlockSpec((8, BN), lambda i, j: (0, i)),
            pl.BlockSpec((BN, 1), lambda i, j: (i, 0)),
            pl.BlockSpec((1, BN, DH), lambda i, j: (j, i, 0)),
        ],
        out_shape=[
            jax.ShapeDtypeStruct((N, D), jnp.float32),
            jax.ShapeDtypeStruct((8, N), jnp.float32),
            jax.ShapeDtypeStruct((N, 1), jnp.float32),
            jax.ShapeDtypeStruct((NC, N, DH), jnp.float32),
        ],
    )(acc, acc, den, h1, aux1, wa1, b1, W2, as2v, ad2v)


def _tc_final(acc, den, h2, aux2, la, wa2, b2):
    def body(alo_ref, ahi_ref, den_ref, h_ref, aux_ref, la_ref, wa_ref, b_ref,
             out_ref):
        asum = den_ref[:, 0:1]
        la = la_ref[...]
        aself = _self_alpha(aux_ref, wa_ref, la)
        dentot = asum + aself + jnp.float32(1e-16)
        accsum = jnp.concatenate([alo_ref[0], ahi_ref[0]], axis=-1)
        out_ref[...] = (accsum + aself * h_ref[...]) / dentot + b_ref[...]

    return pl.pallas_call(
        body,
        grid=(GRID_N,),
        in_specs=[
            pl.BlockSpec((1, BN, DH), lambda i: (0, i, 0)),
            pl.BlockSpec((1, BN, DH), lambda i: (1, i, 0)),
            pl.BlockSpec((BN, 16), lambda i: (i, 0)),
            pl.BlockSpec((BN, D), lambda i: (i, 0)),
            pl.BlockSpec((8, BN), lambda i: (0, i)),
            pl.BlockSpec((BN, 1), lambda i: (i, 0)),
            pl.BlockSpec((2, D), lambda i: (0, 0)),
            pl.BlockSpec((1, D), lambda i: (0, 0)),
        ],
        out_specs=pl.BlockSpec((BN, D), lambda i: (i, 0)),
        out_shape=jax.ShapeDtypeStruct((N, D), jnp.float32),
    )(acc, acc, den, h2, aux2, la, wa2, b2)


def kernel(x, edge_index, edge_weight, W1, We1, as1, ad1, ae1, b1,
           W2, We2, as2, ad2, ae2, b2):
    src = edge_index[0]
    dst = edge_index[1]
    ew = edge_weight[:, 0]
    as1v = as1.reshape(1, D)
    ad1v = ad1.reshape(1, D)
    wa1 = jnp.concatenate([We1.reshape(1, D), ae1.reshape(1, D)], axis=0)
    as2v = as2.reshape(1, D)
    ad2v = ad2.reshape(1, D)
    wa2 = jnp.concatenate([We2.reshape(1, D), ae2.reshape(1, D)], axis=0)
    b1r = b1.reshape(1, D)
    b2r = b2.reshape(1, D)
    z64 = jnp.zeros((RPT, DH), jnp.float32)
    z16 = jnp.zeros((RPT, 16), jnp.float32)

    h1, aux1, h1s = _tc_embed(x, W1, as1v, ad1v)
    accf, den1 = _sc_edge_pass(h1s.reshape(NC * N, DH), aux1, src, dst, ew,
                               wa1, z64, z16)
    acc1 = accf.reshape(NC, N, DH)
    h2, aux2, la, h2s = _tc_combine1(acc1, den1, h1, aux1, wa1, b1r, W2,
                                     as2v, ad2v)
    accf2, den2 = _sc_edge_pass(h2s.reshape(NC * N, DH), aux2, src, dst, ew,
                                wa2, z64, z16)
    acc2 = accf2.reshape(NC, N, DH)
    return _tc_final(acc2, den2, h2, aux2, la, wa2, b2r)


# speedup vs baseline: 28.4662x; 2.3474x over previous
import functools

import jax
import jax.numpy as jnp
from jax import lax
from jax.experimental import pallas as pl
from jax.experimental.pallas import tpu as pltpu
from jax.experimental.pallas import tpu_sc as plsc

N = 10000
E = 320000
D = 128
DH = D // 2
NC = 2
NS = 16
EPT = E // NS
BE = 80
NBATCH = EPT // BE
RPT = 624
NTAIL = N - NS * RPT
BN = 512
GRID_N = (N + BN - 1) // BN


def _sc_edge_pass(hflat, aux, pk, wa, z64, z16):
    mesh = plsc.VectorSubcoreMesh(core_axis_name="c", subcore_axis_name="s")

    @functools.partial(
        pl.kernel,
        out_type=(
            jax.ShapeDtypeStruct((NC * N, DH), jnp.float32),
            jax.ShapeDtypeStruct((N, 16), jnp.float32),
        ),
        mesh=mesh,
        compiler_params=pltpu.CompilerParams(needs_layout_passes=False,
                                             use_tc_tiling_on_sc=False),
        scratch_types=[
            pltpu.VMEM_SHARED((N, DH), jnp.float32),
            pltpu.VMEM_SHARED((N, 16), jnp.float32),
            pltpu.VMEM((N,), jnp.float32),
            pltpu.VMEM((N,), jnp.float32),
            pltpu.VMEM((2, D), jnp.float32),
            pltpu.VMEM((2, 3 * BE), jnp.float32),
            pltpu.VMEM((2, BE), jnp.int32),
            pltpu.VMEM((2, BE), jnp.int32),
            pltpu.VMEM((2, BE), jnp.float32),
            pltpu.VMEM((2, BE, DH), jnp.float32),
            pltpu.VMEM((2, BE, 16), jnp.float32),
            pltpu.VMEM((16,), jnp.float32),
            pltpu.SemaphoreType.DMA,
            pltpu.SemaphoreType.DMA,
            pltpu.SemaphoreType.DMA,
            pltpu.SemaphoreType.DMA,
        ],
    )
    def k(h_hbm, aux_hbm, pk_hbm, wa_hbm, z64_hbm, z16_hbm,
          acc_out, den_out,
          acc_sh, den_sh, asrc_t, adst_t, wa_t, pk_v, dsti_v, gofs_v,
          alpha_v, rows_v, tails_v, red_v, sem_pk, sem_g, sem_sa, sem_sd):
        c = lax.axis_index("c")
        s = lax.axis_index("s")
        pltpu.sync_copy(aux_hbm.at[0], asrc_t)
        pltpu.sync_copy(aux_hbm.at[1], adst_t)
        pltpu.sync_copy(wa_hbm, wa_t)
        pltpu.sync_copy(z16_hbm.at[pl.ds(0, BE)], tails_v.at[0])
        pltpu.sync_copy(z16_hbm.at[pl.ds(0, BE)], tails_v.at[1])
        row0 = pl.multiple_of(s * RPT, 8)
        pltpu.sync_copy(z64_hbm, acc_sh.at[pl.ds(row0, RPT)])

        @pl.when(s == NS - 1)
        def _init_tail():
            pltpu.sync_copy(z64_hbm.at[pl.ds(0, NTAIL)],
                            acc_sh.at[pl.ds(NS * RPT, NTAIL)])

        @pl.when(c == 0)
        def _init_den():
            pltpu.sync_copy(z16_hbm, den_sh.at[pl.ds(row0, RPT)])

            @pl.when(s == NS - 1)
            def _init_den_tail():
                pltpu.sync_copy(z16_hbm.at[pl.ds(0, NTAIL)],
                                den_sh.at[pl.ds(NS * RPT, NTAIL)])

        lane16 = lax.iota(jnp.int32, 16)
        cev = jnp.zeros((16,), jnp.float32)
        for kk in range(D // 16):
            cev = cev + wa_t[0, pl.ds(kk * 16, 16)] * wa_t[1, pl.ds(kk * 16, 16)]
        for shift in (8, 4, 2, 1):
            red_v[...] = cev
            cev = cev + plsc.load_gather(red_v, [lane16 ^ shift])
        ce = cev
        col0 = jnp.zeros((16,), jnp.int32)
        col1 = col0 + 1
        col2 = col0 + 2
        ones16 = jnp.ones((16,), jnp.float32)
        gofs0 = c * N
        plsc.subcore_barrier()
        row_base = s * NBATCH

        def ld_pk(b, p):
            return pltpu.make_async_copy(pk_hbm.at[row_base + b],
                                         pk_v.at[p], sem_pk)

        def gather(p):
            return pltpu.make_async_copy(h_hbm.at[gofs_v.at[p]],
                                         rows_v.at[p], sem_g)

        def sc_acc(p):
            return pltpu.make_async_copy(rows_v.at[p],
                                         acc_sh.at[dsti_v.at[p]], sem_sa)

        def sc_den(p):
            return pltpu.make_async_copy(tails_v.at[p],
                                         den_sh.at[dsti_v.at[p]], sem_sd)

        def scalar_phase(p):
            for g in range(BE // 16):
                off = g * 16
                si = plsc.bitcast(pk_v[p, pl.ds(off, 16)], jnp.int32)
                di = plsc.bitcast(pk_v[p, pl.ds(BE + off, 16)], jnp.int32)
                ew16 = pk_v[p, pl.ds(2 * BE + off, 16)]
                a = (plsc.load_gather(asrc_t, [si])
                     + plsc.load_gather(adst_t, [di]) + ce * ew16)
                a = jnp.where(a > 0, a, a * jnp.float32(0.2))
                a = jnp.exp(a)
                alpha_v[p, pl.ds(off, 16)] = a
                gofs_v[p, pl.ds(off, 16)] = si + gofs0
                dsti_v[p, pl.ds(off, 16)] = di
                bidx = jnp.broadcast_to(p, (16,))
                row16 = off + lane16
                plsc.store_scatter(tails_v, [bidx, row16, col0], a)
                plsc.store_scatter(tails_v, [bidx, row16, col1], ones16)
                plsc.store_scatter(tails_v, [bidx, row16, col2], ew16)

        def multiply(p):
            def row_body(r, carry2):
                spl = plsc.load_gather(
                    alpha_v, [jnp.broadcast_to(p, (16,)),
                              jnp.broadcast_to(r, (16,))])
                for cc in range(DH // 16):
                    v = rows_v[p, r, pl.ds(cc * 16, 16)]
                    rows_v[p, r, pl.ds(cc * 16, 16)] = v * spl
                return carry2

            lax.fori_loop(0, BE, row_body, 0, unroll=2)

        def start_scatters(p):
            sc_acc(p).start(add=True)

            @pl.when(c == 0)
            def _():
                sc_den(p).start(add=True)

        def wait_scatters(p):
            sc_acc(p).wait()

            @pl.when(c == 0)
            def _():
                sc_den(p).wait()

        pltpu.sync_copy(pk_hbm.at[row_base], pk_v.at[0])
        scalar_phase(0)
        gather(0).start()
        ld_pk(1, 1).start()

        def batch_body(b, carry):
            p = lax.bitwise_and(b, 1)
            q = 1 - p
            ld_pk(b, p).wait()

            @pl.when(b >= 2)
            def _():
                wait_scatters(p)

            scalar_phase(p)
            gather(p).start()

            @pl.when(b < NBATCH - 1)
            def _():
                ld_pk(b + 1, q).start()

            gather(q).wait()
            multiply(q)
            start_scatters(q)
            return carry

        lax.fori_loop(1, NBATCH, batch_body, 0)
        p_last = (NBATCH - 1) & 1
        gather(p_last).wait()
        multiply(p_last)
        start_scatters(p_last)
        wait_scatters(1 - p_last)
        wait_scatters(p_last)
        plsc.subcore_barrier()
        out0 = pl.multiple_of(c * N + row0, 8)
        pltpu.sync_copy(acc_sh.at[pl.ds(row0, RPT)],
                        acc_out.at[pl.ds(out0, RPT)])

        @pl.when(c == 0)
        def _out_den():
            pltpu.sync_copy(den_sh.at[pl.ds(row0, RPT)],
                            den_out.at[pl.ds(row0, RPT)])

        @pl.when(s == NS - 1)
        def _out_tail():
            t0 = pl.multiple_of(c * N + NS * RPT, 8)
            pltpu.sync_copy(acc_sh.at[pl.ds(NS * RPT, NTAIL)],
                            acc_out.at[pl.ds(t0, NTAIL)])

            @pl.when(c == 0)
            def _out_den_tail():
                pltpu.sync_copy(den_sh.at[pl.ds(NS * RPT, NTAIL)],
                                den_out.at[pl.ds(NS * RPT, NTAIL)])

    return k(hflat, aux, pk, wa, z64, z16)


def _tc_pack(src2d, dst2d, ew2d):
    EB = E // BE

    def body(s_ref, d_ref, e_ref, out_ref):
        sf = lax.bitcast_convert_type(s_ref[...], jnp.float32)
        df = lax.bitcast_convert_type(d_ref[...], jnp.float32)
        out_ref[...] = jnp.concatenate([sf, df, e_ref[...]], axis=1)

    blk = 200
    return pl.pallas_call(
        body,
        grid=(EB // blk,),
        in_specs=[
            pl.BlockSpec((blk, BE), lambda i: (i, 0)),
            pl.BlockSpec((blk, BE), lambda i: (i, 0)),
            pl.BlockSpec((blk, BE), lambda i: (i, 0)),
        ],
        out_specs=pl.BlockSpec((blk, 3 * BE), lambda i: (i, 0)),
        out_shape=jax.ShapeDtypeStruct((EB, 3 * BE), jnp.float32),
    )(src2d, dst2d, ew2d)


def _dot_t(a, b):
    return lax.dot_general(a, b, (((1,), (1,)), ((), ())),
                           preferred_element_type=jnp.float32)


def _split_cols(h, j):
    return jnp.where(j == 0, h[:, :DH], h[:, DH:])[None]


def _tc_embed(x, W, asv, adv):
    def body(x_ref, w_ref, as_ref, ad_ref, h_ref, aux_ref, hs_ref):
        h = _dot_t(x_ref[...], w_ref[...])
        h_ref[...] = h
        aux_ref[...] = jnp.zeros((8, BN), jnp.float32)
        aux_ref[0:1, :] = _dot_t(as_ref[...], h)
        aux_ref[1:2, :] = _dot_t(ad_ref[...], h)
        hs_ref[...] = _split_cols(h, pl.program_id(1))

    return pl.pallas_call(
        body,
        grid=(GRID_N, NC),
        in_specs=[
            pl.BlockSpec((BN, D), lambda i, j: (i, 0)),
            pl.BlockSpec((D, D), lambda i, j: (0, 0)),
            pl.BlockSpec((1, D), lambda i, j: (0, 0)),
            pl.BlockSpec((1, D), lambda i, j: (0, 0)),
        ],
        out_specs=[
            pl.BlockSpec((BN, D), lambda i, j: (i, 0)),
            pl.BlockSpec((8, BN), lambda i, j: (0, i)),
            pl.BlockSpec((1, BN, DH), lambda i, j: (j, i, 0)),
        ],
        out_shape=[
            jax.ShapeDtypeStruct((N, D), jnp.float32),
            jax.ShapeDtypeStruct((8, N), jnp.float32),
            jax.ShapeDtypeStruct((NC, N, DH), jnp.float32),
        ],
    )(x, W, asv, adv)


def _self_alpha(aux_ref, wa_ref, la):
    auxT = lax.dot_general(aux_ref[...], jnp.eye(8, dtype=jnp.float32),
                           (((0,), (0,)), ((), ())),
                           preferred_element_type=jnp.float32)
    wa = wa_ref[...]
    ce = jnp.sum(wa[0:1, :] * wa[1:2, :])
    logit = auxT[:, 0:1] + auxT[:, 1:2] + ce * la
    logit = jnp.where(logit > 0, logit, logit * jnp.float32(0.2))
    return jnp.exp(logit)


def _tc_combine1(acc, den, h1, aux1, wa1, b1, W2, as2v, ad2v):
    def body(alo_ref, ahi_ref, den_ref, h_ref, aux_ref, wa_ref, b_ref, w2_ref,
             as_ref, ad_ref, h2_ref, aux2_ref, la_ref, h2s_ref):
        dsum = den_ref[...]
        asum = dsum[:, 0:1]
        deg = dsum[:, 1:2]
        ews = dsum[:, 2:3]
        la = ews / jnp.maximum(deg, 1.0)
        aself = _self_alpha(aux_ref, wa_ref, la)
        dentot = asum + aself + jnp.float32(1e-16)
        accsum = jnp.concatenate([alo_ref[0], ahi_ref[0]], axis=-1)
        out1 = (accsum + aself * h_ref[...]) / dentot + b_ref[...]
        x2 = jnp.maximum(out1, 0.0)
        h2 = _dot_t(x2, w2_ref[...])
        h2_ref[...] = h2
        aux2_ref[...] = jnp.zeros((8, BN), jnp.float32)
        aux2_ref[0:1, :] = _dot_t(as_ref[...], h2)
        aux2_ref[1:2, :] = _dot_t(ad_ref[...], h2)
        la_ref[...] = la
        h2s_ref[...] = _split_cols(h2, pl.program_id(1))

    return pl.pallas_call(
        body,
        grid=(GRID_N, NC),
        in_specs=[
            pl.BlockSpec((1, BN, DH), lambda i, j: (0, i, 0)),
            pl.BlockSpec((1, BN, DH), lambda i, j: (1, i, 0)),
            pl.BlockSpec((BN, 16), lambda i, j: (i, 0)),
            pl.BlockSpec((BN, D), lambda i, j: (i, 0)),
            pl.BlockSpec((8, BN), lambda i, j: (0, i)),
            pl.BlockSpec((2, D), lambda i, j: (0, 0)),
            pl.BlockSpec((1, D), lambda i, j: (0, 0)),
            pl.BlockSpec((D, D), lambda i, j: (0, 0)),
            pl.BlockSpec((1, D), lambda i, j: (0, 0)),
            pl.BlockSpec((1, D), lambda i, j: (0, 0)),
        ],
        out_specs=[
            pl.BlockSpec((BN, D), lambda i, j: (i, 0)),
            pl.BlockSpec((8, BN), lambda i, j: (0, i)),
            pl.BlockSpec((BN, 1), lambda i, j: (i, 0)),
            pl.BlockSpec((1, BN, DH), lambda i, j: (j, i, 0)),
        ],
        out_shape=[
            jax.ShapeDtypeStruct((N, D), jnp.float32),
            jax.ShapeDtypeStruct((8, N), jnp.float32),
            jax.ShapeDtypeStruct((N, 1), jnp.float32),
            jax.ShapeDtypeStruct((NC, N, DH), jnp.float32),
        ],
    )(acc, acc, den, h1, aux1, wa1, b1, W2, as2v, ad2v)


def _tc_final(acc, den, h2, aux2, la, wa2, b2):
    def body(alo_ref, ahi_ref, den_ref, h_ref, aux_ref, la_ref, wa_ref, b_ref,
             out_ref):
        asum = den_ref[:, 0:1]
        la = la_ref[...]
        aself = _self_alpha(aux_ref, wa_ref, la)
        dentot = asum + aself + jnp.float32(1e-16)
        accsum = jnp.concatenate([alo_ref[0], ahi_ref[0]], axis=-1)
        out_ref[...] = (accsum + aself * h_ref[...]) / dentot + b_ref[...]

    return pl.pallas_call(
        body,
        grid=(GRID_N,),
        in_specs=[
            pl.BlockSpec((1, BN, DH), lambda i: (0, i, 0)),
            pl.BlockSpec((1, BN, DH), lambda i: (1, i, 0)),
            pl.BlockSpec((BN, 16), lambda i: (i, 0)),
            pl.BlockSpec((BN, D), lambda i: (i, 0)),
            pl.BlockSpec((8, BN), lambda i: (0, i)),
            pl.BlockSpec((BN, 1), lambda i: (i, 0)),
            pl.BlockSpec((2, D), lambda i: (0, 0)),
            pl.BlockSpec((1, D), lambda i: (0, 0)),
        ],
        out_specs=pl.BlockSpec((BN, D), lambda i: (i, 0)),
        out_shape=jax.ShapeDtypeStruct((N, D), jnp.float32),
    )(acc, acc, den, h2, aux2, la, wa2, b2)


def kernel(x, edge_index, edge_weight, W1, We1, as1, ad1, ae1, b1,
           W2, We2, as2, ad2, ae2, b2):
    src = edge_index[0]
    dst = edge_index[1]
    ew = edge_weight[:, 0]
    as1v = as1.reshape(1, D)
    ad1v = ad1.reshape(1, D)
    wa1 = jnp.concatenate([We1.reshape(1, D), ae1.reshape(1, D)], axis=0)
    as2v = as2.reshape(1, D)
    ad2v = ad2.reshape(1, D)
    wa2 = jnp.concatenate([We2.reshape(1, D), ae2.reshape(1, D)], axis=0)
    b1r = b1.reshape(1, D)
    b2r = b2.reshape(1, D)
    z64 = jnp.zeros((RPT, DH), jnp.float32)
    z16 = jnp.zeros((RPT, 16), jnp.float32)

    pk = _tc_pack(src.reshape(E // BE, BE), dst.reshape(E // BE, BE),
                  ew.reshape(E // BE, BE))
    h1, aux1, h1s = _tc_embed(x, W1, as1v, ad1v)
    accf, den1 = _sc_edge_pass(h1s.reshape(NC * N, DH), aux1, pk,
                               wa1, z64, z16)
    acc1 = accf.reshape(NC, N, DH)
    h2, aux2, la, h2s = _tc_combine1(acc1, den1, h1, aux1, wa1, b1r, W2,
                                     as2v, ad2v)
    accf2, den2 = _sc_edge_pass(h2s.reshape(NC * N, DH), aux2, pk,
                                wa2, z64, z16)
    acc2 = accf2.reshape(NC, N, DH)
    return _tc_final(acc2, den2, h2, aux2, la, wa2, b2r)


# speedup vs baseline: 28.9941x; 1.0185x over previous
import functools

import jax
import jax.numpy as jnp
from jax import lax
from jax.experimental import pallas as pl
from jax.experimental.pallas import tpu as pltpu
from jax.experimental.pallas import tpu_sc as plsc

N = 10000
E = 320000
D = 128
DH = D // 2
NC = 2
NS = 16
EPT = E // NS
BE = 80
NBATCH = EPT // BE
RPT = 624
NTAIL = N - NS * RPT
BN = 512
GRID_N = (N + BN - 1) // BN


def _sc_edge_pass(hflat, aux, pk, wa, z64, z16):
    mesh = plsc.VectorSubcoreMesh(core_axis_name="c", subcore_axis_name="s")

    @functools.partial(
        pl.kernel,
        out_type=(
            jax.ShapeDtypeStruct((NC * N, DH), jnp.float32),
            jax.ShapeDtypeStruct((N, 16), jnp.float32),
        ),
        mesh=mesh,
        compiler_params=pltpu.CompilerParams(needs_layout_passes=False,
                                             use_tc_tiling_on_sc=False),
        scratch_types=[
            pltpu.VMEM_SHARED((N, DH), jnp.float32),
            pltpu.VMEM_SHARED((N, 16), jnp.float32),
            pltpu.VMEM((N,), jnp.float32),
            pltpu.VMEM((N,), jnp.float32),
            pltpu.VMEM((2, D), jnp.float32),
            pltpu.VMEM((2, 3 * BE), jnp.float32),
            pltpu.VMEM((2, BE), jnp.int32),
            pltpu.VMEM((2, BE), jnp.int32),
            pltpu.VMEM((2, BE), jnp.float32),
            pltpu.VMEM((2, BE, DH), jnp.float32),
            pltpu.VMEM((2, BE, 16), jnp.float32),
            pltpu.VMEM((16,), jnp.float32),
            pltpu.SemaphoreType.DMA,
            pltpu.SemaphoreType.DMA,
            pltpu.SemaphoreType.DMA,
            pltpu.SemaphoreType.DMA,
        ],
    )
    def k(h_hbm, aux_hbm, pk_hbm, wa_hbm, z64_hbm, z16_hbm,
          acc_out, den_out,
          acc_sh, den_sh, asrc_t, adst_t, wa_t, pk_v, dsti_v, gofs_v,
          alpha_v, rows_v, tails_v, red_v, sem_pk, sem_g, sem_sa, sem_sd):
        c = lax.axis_index("c")
        s = lax.axis_index("s")
        pltpu.sync_copy(aux_hbm.at[0], asrc_t)
        pltpu.sync_copy(aux_hbm.at[1], adst_t)
        pltpu.sync_copy(wa_hbm, wa_t)
        pltpu.sync_copy(z16_hbm.at[pl.ds(0, BE)], tails_v.at[0])
        pltpu.sync_copy(z16_hbm.at[pl.ds(0, BE)], tails_v.at[1])
        row0 = pl.multiple_of(s * RPT, 8)
        pltpu.sync_copy(z64_hbm, acc_sh.at[pl.ds(row0, RPT)])

        @pl.when(s == NS - 1)
        def _init_tail():
            pltpu.sync_copy(z64_hbm.at[pl.ds(0, NTAIL)],
                            acc_sh.at[pl.ds(NS * RPT, NTAIL)])

        @pl.when(c == 0)
        def _init_den():
            pltpu.sync_copy(z16_hbm, den_sh.at[pl.ds(row0, RPT)])

            @pl.when(s == NS - 1)
            def _init_den_tail():
                pltpu.sync_copy(z16_hbm.at[pl.ds(0, NTAIL)],
                                den_sh.at[pl.ds(NS * RPT, NTAIL)])

        lane16 = lax.iota(jnp.int32, 16)
        cev = jnp.zeros((16,), jnp.float32)
        for kk in range(D // 16):
            cev = cev + wa_t[0, pl.ds(kk * 16, 16)] * wa_t[1, pl.ds(kk * 16, 16)]
        for shift in (8, 4, 2, 1):
            red_v[...] = cev
            cev = cev + plsc.load_gather(red_v, [lane16 ^ shift])
        ce = cev
        col0 = jnp.zeros((16,), jnp.int32)
        col1 = col0 + 1
        col2 = col0 + 2
        ones16 = jnp.ones((16,), jnp.float32)
        gofs0 = c * N
        plsc.subcore_barrier()
        row_base = s * NBATCH

        def ld_pk(b, p):
            return pltpu.make_async_copy(pk_hbm.at[row_base + b],
                                         pk_v.at[p], sem_pk)

        def gather(p):
            return pltpu.make_async_copy(h_hbm.at[gofs_v.at[p]],
                                         rows_v.at[p], sem_g)

        def sc_acc(p):
            return pltpu.make_async_copy(rows_v.at[p],
                                         acc_sh.at[dsti_v.at[p]], sem_sa)

        def sc_den(p):
            return pltpu.make_async_copy(tails_v.at[p],
                                         den_sh.at[dsti_v.at[p]], sem_sd)

        def scalar_phase(p):
            for g in range(BE // 16):
                off = g * 16
                si = plsc.bitcast(pk_v[p, pl.ds(off, 16)], jnp.int32)
                di = plsc.bitcast(pk_v[p, pl.ds(BE + off, 16)], jnp.int32)
                ew16 = pk_v[p, pl.ds(2 * BE + off, 16)]
                a = (plsc.load_gather(asrc_t, [si])
                     + plsc.load_gather(adst_t, [di]) + ce * ew16)
                a = jnp.where(a > 0, a, a * jnp.float32(0.2))
                a = jnp.exp(a)
                alpha_v[p, pl.ds(off, 16)] = a
                gofs_v[p, pl.ds(off, 16)] = si + gofs0
                dsti_v[p, pl.ds(off, 16)] = di
                bidx = jnp.broadcast_to(p, (16,))
                row16 = off + lane16
                plsc.store_scatter(tails_v, [bidx, row16, col0], a)
                plsc.store_scatter(tails_v, [bidx, row16, col1], ones16)
                plsc.store_scatter(tails_v, [bidx, row16, col2], ew16)

        def multiply(p):
            def row_body(r, carry2):
                spl = plsc.load_gather(
                    alpha_v, [jnp.broadcast_to(p, (16,)),
                              jnp.broadcast_to(r, (16,))])
                for cc in range(DH // 16):
                    v = rows_v[p, r, pl.ds(cc * 16, 16)]
                    rows_v[p, r, pl.ds(cc * 16, 16)] = v * spl
                return carry2

            lax.fori_loop(0, BE, row_body, 0, unroll=4)

        def start_scatters(p):
            sc_acc(p).start(add=True)

            @pl.when(c == 0)
            def _():
                sc_den(p).start(add=True)

        def wait_scatters(p):
            sc_acc(p).wait()

            @pl.when(c == 0)
            def _():
                sc_den(p).wait()

        pltpu.sync_copy(pk_hbm.at[row_base], pk_v.at[0])
        scalar_phase(0)
        gather(0).start()
        ld_pk(1, 1).start()

        def batch_body(b, carry):
            p = lax.bitwise_and(b, 1)
            q = 1 - p
            ld_pk(b, p).wait()

            @pl.when(b >= 2)
            def _():
                wait_scatters(p)

            scalar_phase(p)
            gather(p).start()

            @pl.when(b < NBATCH - 1)
            def _():
                ld_pk(b + 1, q).start()

            gather(q).wait()
            multiply(q)
            start_scatters(q)
            return carry

        lax.fori_loop(1, NBATCH, batch_body, 0)
        p_last = (NBATCH - 1) & 1
        gather(p_last).wait()
        multiply(p_last)
        start_scatters(p_last)
        wait_scatters(1 - p_last)
        wait_scatters(p_last)
        plsc.subcore_barrier()
        out0 = pl.multiple_of(c * N + row0, 8)
        pltpu.sync_copy(acc_sh.at[pl.ds(row0, RPT)],
                        acc_out.at[pl.ds(out0, RPT)])

        @pl.when(c == 0)
        def _out_den():
            pltpu.sync_copy(den_sh.at[pl.ds(row0, RPT)],
                            den_out.at[pl.ds(row0, RPT)])

        @pl.when(s == NS - 1)
        def _out_tail():
            t0 = pl.multiple_of(c * N + NS * RPT, 8)
            pltpu.sync_copy(acc_sh.at[pl.ds(NS * RPT, NTAIL)],
                            acc_out.at[pl.ds(t0, NTAIL)])

            @pl.when(c == 0)
            def _out_den_tail():
                pltpu.sync_copy(den_sh.at[pl.ds(NS * RPT, NTAIL)],
                                den_out.at[pl.ds(NS * RPT, NTAIL)])

    return k(hflat, aux, pk, wa, z64, z16)


def _tc_pack(src2d, dst2d, ew2d):
    EB = E // BE

    def body(s_ref, d_ref, e_ref, out_ref):
        sf = lax.bitcast_convert_type(s_ref[...], jnp.float32)
        df = lax.bitcast_convert_type(d_ref[...], jnp.float32)
        out_ref[...] = jnp.concatenate([sf, df, e_ref[...]], axis=1)

    blk = 200
    return pl.pallas_call(
        body,
        grid=(EB // blk,),
        in_specs=[
            pl.BlockSpec((blk, BE), lambda i: (i, 0)),
            pl.BlockSpec((blk, BE), lambda i: (i, 0)),
            pl.BlockSpec((blk, BE), lambda i: (i, 0)),
        ],
        out_specs=pl.BlockSpec((blk, 3 * BE), lambda i: (i, 0)),
        out_shape=jax.ShapeDtypeStruct((EB, 3 * BE), jnp.float32),
    )(src2d, dst2d, ew2d)


def _dot_t(a, b):
    return lax.dot_general(a, b, (((1,), (1,)), ((), ())),
                           preferred_element_type=jnp.float32)


def _split_cols(h, j):
    return jnp.where(j == 0, h[:, :DH], h[:, DH:])[None]


def _tc_embed(x, W, asv, adv):
    def body(x_ref, w_ref, as_ref, ad_ref, h_ref, aux_ref, hs_ref):
        h = _dot_t(x_ref[...], w_ref[...])
        h_ref[...] = h
        aux_ref[...] = jnp.zeros((8, BN), jnp.float32)
        aux_ref[0:1, :] = _dot_t(as_ref[...], h)
        aux_ref[1:2, :] = _dot_t(ad_ref[...], h)
        hs_ref[...] = _split_cols(h, pl.program_id(1))

    return pl.pallas_call(
        body,
        grid=(GRID_N, NC),
        in_specs=[
            pl.BlockSpec((BN, D), lambda i, j: (i, 0)),
            pl.BlockSpec((D, D), lambda i, j: (0, 0)),
            pl.BlockSpec((1, D), lambda i, j: (0, 0)),
            pl.BlockSpec((1, D), lambda i, j: (0, 0)),
        ],
        out_specs=[
            pl.BlockSpec((BN, D), lambda i, j: (i, 0)),
            pl.BlockSpec((8, BN), lambda i, j: (0, i)),
            pl.BlockSpec((1, BN, DH), lambda i, j: (j, i, 0)),
        ],
        out_shape=[
            jax.ShapeDtypeStruct((N, D), jnp.float32),
            jax.ShapeDtypeStruct((8, N), jnp.float32),
            jax.ShapeDtypeStruct((NC, N, DH), jnp.float32),
        ],
    )(x, W, asv, adv)


def _self_alpha(aux_ref, wa_ref, la):
    auxT = lax.dot_general(aux_ref[...], jnp.eye(8, dtype=jnp.float32),
                           (((0,), (0,)), ((), ())),
                           preferred_element_type=jnp.float32)
    wa = wa_ref[...]
    ce = jnp.sum(wa[0:1, :] * wa[1:2, :])
    logit = auxT[:, 0:1] + auxT[:, 1:2] + ce * la
    logit = jnp.where(logit > 0, logit, logit * jnp.float32(0.2))
    return jnp.exp(logit)


def _tc_combine1(acc, den, h1, aux1, wa1, b1, W2, as2v, ad2v):
    def body(alo_ref, ahi_ref, den_ref, h_ref, aux_ref, wa_ref, b_ref, w2_ref,
             as_ref, ad_ref, h2_ref, aux2_ref, la_ref, h2s_ref):
        dsum = den_ref[...]
        asum = dsum[:, 0:1]
        deg = dsum[:, 1:2]
        ews = dsum[:, 2:3]
        la = ews / jnp.maximum(deg, 1.0)
        aself = _self_alpha(aux_ref, wa_ref, la)
        dentot = asum + aself + jnp.float32(1e-16)
        accsum = jnp.concatenate([alo_ref[0], ahi_ref[0]], axis=-1)
        out1 = (accsum + aself * h_ref[...]) / dentot + b_ref[...]
        x2 = jnp.maximum(out1, 0.0)
        h2 = _dot_t(x2, w2_ref[...])
        h2_ref[...] = h2
        aux2_ref[...] = jnp.zeros((8, BN), jnp.float32)
        aux2_ref[0:1, :] = _dot_t(as_ref[...], h2)
        aux2_ref[1:2, :] = _dot_t(ad_ref[...], h2)
        la_ref[...] = la
        h2s_ref[...] = _split_cols(h2, pl.program_id(1))

    return pl.pallas_call(
        body,
        grid=(GRID_N, NC),
        in_specs=[
            pl.BlockSpec((1, BN, DH), lambda i, j: (0, i, 0)),
            pl.BlockSpec((1, BN, DH), lambda i, j: (1, i, 0)),
            pl.BlockSpec((BN, 16), lambda i, j: (i, 0)),
            pl.BlockSpec((BN, D), lambda i, j: (i, 0)),
            pl.BlockSpec((8, BN), lambda i, j: (0, i)),
            pl.BlockSpec((2, D), lambda i, j: (0, 0)),
            pl.BlockSpec((1, D), lambda i, j: (0, 0)),
            pl.BlockSpec((D, D), lambda i, j: (0, 0)),
            pl.BlockSpec((1, D), lambda i, j: (0, 0)),
            pl.BlockSpec((1, D), lambda i, j: (0, 0)),
        ],
        out_specs=[
            pl.BlockSpec((BN, D), lambda i, j: (i, 0)),
            pl.BlockSpec((8, BN), lambda i, j: (0, i)),
            pl.BlockSpec((BN, 1), lambda i, j: (i, 0)),
            pl.BlockSpec((1, BN, DH), lambda i, j: (j, i, 0)),
        ],
        out_shape=[
            jax.ShapeDtypeStruct((N, D), jnp.float32),
            jax.ShapeDtypeStruct((8, N), jnp.float32),
            jax.ShapeDtypeStruct((N, 1), jnp.float32),
            jax.ShapeDtypeStruct((NC, N, DH), jnp.float32),
        ],
    )(acc, acc, den, h1, aux1, wa1, b1, W2, as2v, ad2v)


def _tc_final(acc, den, h2, aux2, la, wa2, b2):
    def body(alo_ref, ahi_ref, den_ref, h_ref, aux_ref, la_ref, wa_ref, b_ref,
             out_ref):
        asum = den_ref[:, 0:1]
        la = la_ref[...]
        aself = _self_alpha(aux_ref, wa_ref, la)
        dentot = asum + aself + jnp.float32(1e-16)
        accsum = jnp.concatenate([alo_ref[0], ahi_ref[0]], axis=-1)
        out_ref[...] = (accsum + aself * h_ref[...]) / dentot + b_ref[...]

    return pl.pallas_call(
        body,
        grid=(GRID_N,),
        in_specs=[
            pl.BlockSpec((1, BN, DH), lambda i: (0, i, 0)),
            pl.BlockSpec((1, BN, DH), lambda i: (1, i, 0)),
            pl.BlockSpec((BN, 16), lambda i: (i, 0)),
            pl.BlockSpec((BN, D), lambda i: (i, 0)),
            pl.BlockSpec((8, BN), lambda i: (0, i)),
            pl.BlockSpec((BN, 1), lambda i: (i, 0)),
            pl.BlockSpec((2, D), lambda i: (0, 0)),
            pl.BlockSpec((1, D), lambda i: (0, 0)),
        ],
        out_specs=pl.BlockSpec((BN, D), lambda i: (i, 0)),
        out_shape=jax.ShapeDtypeStruct((N, D), jnp.float32),
    )(acc, acc, den, h2, aux2, la, wa2, b2)


def kernel(x, edge_index, edge_weight, W1, We1, as1, ad1, ae1, b1,
           W2, We2, as2, ad2, ae2, b2):
    src = edge_index[0]
    dst = edge_index[1]
    ew = edge_weight[:, 0]
    as1v = as1.reshape(1, D)
    ad1v = ad1.reshape(1, D)
    wa1 = jnp.concatenate([We1.reshape(1, D), ae1.reshape(1, D)], axis=0)
    as2v = as2.reshape(1, D)
    ad2v = ad2.reshape(1, D)
    wa2 = jnp.concatenate([We2.reshape(1, D), ae2.reshape(1, D)], axis=0)
    b1r = b1.reshape(1, D)
    b2r = b2.reshape(1, D)
    z64 = jnp.zeros((RPT, DH), jnp.float32)
    z16 = jnp.zeros((RPT, 16), jnp.float32)

    pk = _tc_pack(src.reshape(E // BE, BE), dst.reshape(E // BE, BE),
                  ew.reshape(E // BE, BE))
    h1, aux1, h1s = _tc_embed(x, W1, as1v, ad1v)
    accf, den1 = _sc_edge_pass(h1s.reshape(NC * N, DH), aux1, pk,
                               wa1, z64, z16)
    acc1 = accf.reshape(NC, N, DH)
    h2, aux2, la, h2s = _tc_combine1(acc1, den1, h1, aux1, wa1, b1r, W2,
                                     as2v, ad2v)
    accf2, den2 = _sc_edge_pass(h2s.reshape(NC * N, DH), aux2, pk,
                                wa2, z64, z16)
    acc2 = accf2.reshape(NC, N, DH)
    return _tc_final(acc2, den2, h2, aux2, la, wa2, b2r)


# speedup vs baseline: 29.9078x; 1.0315x over previous
import functools

import jax
import jax.numpy as jnp
from jax import lax
from jax.experimental import pallas as pl
from jax.experimental.pallas import tpu as pltpu
from jax.experimental.pallas import tpu_sc as plsc

N = 10000
E = 320000
D = 128
DH = D // 2
NC = 2
NS = 16
EPT = E // NS
BE = 80
NBATCH = EPT // BE
RPT = 624
NTAIL = N - NS * RPT
BN = 512
NBLK = 20
NPAD = BN * NBLK


def _sc_edge_pass(hflat, aux, src, dst, ew, wa, z64, z16):
    mesh = plsc.VectorSubcoreMesh(core_axis_name="c", subcore_axis_name="s")

    @functools.partial(
        pl.kernel,
        out_type=(
            jax.ShapeDtypeStruct((NC * NPAD, DH), jnp.float32),
            jax.ShapeDtypeStruct((N, 16), jnp.float32),
        ),
        mesh=mesh,
        compiler_params=pltpu.CompilerParams(needs_layout_passes=False,
                                             use_tc_tiling_on_sc=False),
        scratch_types=[
            pltpu.VMEM_SHARED((N, DH), jnp.float32),
            pltpu.VMEM_SHARED((N, 16), jnp.float32),
            pltpu.VMEM((N,), jnp.float32),
            pltpu.VMEM((N,), jnp.float32),
            pltpu.VMEM((2, D), jnp.float32),
            pltpu.VMEM((2, BE), jnp.int32),
            pltpu.VMEM((2, BE), jnp.int32),
            pltpu.VMEM((2, BE), jnp.float32),
            pltpu.VMEM((2, BE), jnp.int32),
            pltpu.VMEM((2, BE), jnp.float32),
            pltpu.VMEM((2, BE, DH), jnp.float32),
            pltpu.VMEM((2, BE, 16), jnp.float32),
            pltpu.VMEM((16,), jnp.float32),
            pltpu.SemaphoreType.DMA,
            pltpu.SemaphoreType.DMA,
            pltpu.SemaphoreType.DMA,
            pltpu.SemaphoreType.DMA,
        ],
    )
    def k(h_hbm, aux_hbm, src_hbm, dst_hbm, ew_hbm, wa_hbm, z64_hbm, z16_hbm,
          acc_out, den_out,
          acc_sh, den_sh, asrc_t, adst_t, wa_t, srcb_v, dstb_v, ewb_v,
          gofs_v, alpha_v, rows_v, tails_v, red_v,
          sem_pk, sem_g, sem_sa, sem_sd):
        c = lax.axis_index("c")
        s = lax.axis_index("s")
        pltpu.sync_copy(aux_hbm.at[0], asrc_t)
        pltpu.sync_copy(aux_hbm.at[1], adst_t)
        pltpu.sync_copy(wa_hbm, wa_t)
        pltpu.sync_copy(z16_hbm.at[pl.ds(0, BE)], tails_v.at[0])
        pltpu.sync_copy(z16_hbm.at[pl.ds(0, BE)], tails_v.at[1])
        row0 = pl.multiple_of(s * RPT, 8)
        pltpu.sync_copy(z64_hbm, acc_sh.at[pl.ds(row0, RPT)])

        @pl.when(s == NS - 1)
        def _init_tail():
            pltpu.sync_copy(z64_hbm.at[pl.ds(0, NTAIL)],
                            acc_sh.at[pl.ds(NS * RPT, NTAIL)])

        @pl.when(c == 0)
        def _init_den():
            pltpu.sync_copy(z16_hbm, den_sh.at[pl.ds(row0, RPT)])

            @pl.when(s == NS - 1)
            def _init_den_tail():
                pltpu.sync_copy(z16_hbm.at[pl.ds(0, NTAIL)],
                                den_sh.at[pl.ds(NS * RPT, NTAIL)])

        lane16 = lax.iota(jnp.int32, 16)
        cev = jnp.zeros((16,), jnp.float32)
        for kk in range(D // 16):
            cev = cev + wa_t[0, pl.ds(kk * 16, 16)] * wa_t[1, pl.ds(kk * 16, 16)]
        for shift in (8, 4, 2, 1):
            red_v[...] = cev
            cev = cev + plsc.load_gather(red_v, [lane16 ^ shift])
        ce = cev
        col0 = jnp.zeros((16,), jnp.int32)
        col1 = col0 + 1
        col2 = col0 + 2
        ones16 = jnp.ones((16,), jnp.float32)
        gofs0 = c * NPAD
        plsc.subcore_barrier()
        ebase0 = s * EPT

        def ld_idx(b, p):
            eb = pl.multiple_of(ebase0 + b * BE, 8)
            return (pltpu.make_async_copy(src_hbm.at[pl.ds(eb, BE)],
                                          srcb_v.at[p], sem_pk),
                    pltpu.make_async_copy(dst_hbm.at[pl.ds(eb, BE)],
                                          dstb_v.at[p], sem_pk),
                    pltpu.make_async_copy(ew_hbm.at[pl.ds(eb, BE)],
                                          ewb_v.at[p], sem_pk))

        def start_idx(b, p):
            for d in ld_idx(b, p):
                d.start()

        def wait_idx(b, p):
            for d in ld_idx(b, p):
                d.wait()

        def gather(p):
            return pltpu.make_async_copy(h_hbm.at[gofs_v.at[p]],
                                         rows_v.at[p], sem_g)

        def sc_acc(p):
            return pltpu.make_async_copy(rows_v.at[p],
                                         acc_sh.at[dstb_v.at[p]], sem_sa)

        def sc_den(p):
            return pltpu.make_async_copy(tails_v.at[p],
                                         den_sh.at[dstb_v.at[p]], sem_sd)

        def scalar_phase(p):
            for g in range(BE // 16):
                off = g * 16
                si = srcb_v[p, pl.ds(off, 16)]
                di = dstb_v[p, pl.ds(off, 16)]
                ew16 = ewb_v[p, pl.ds(off, 16)]
                a = (plsc.load_gather(asrc_t, [si])
                     + plsc.load_gather(adst_t, [di]) + ce * ew16)
                a = jnp.where(a > 0, a, a * jnp.float32(0.2))
                a = jnp.exp(a)
                alpha_v[p, pl.ds(off, 16)] = a
                gofs_v[p, pl.ds(off, 16)] = si + gofs0
                bidx = jnp.broadcast_to(p, (16,))
                row16 = off + lane16
                plsc.store_scatter(tails_v, [bidx, row16, col0], a)
                plsc.store_scatter(tails_v, [bidx, row16, col1], ones16)
                plsc.store_scatter(tails_v, [bidx, row16, col2], ew16)

        def multiply(p):
            def row_body(r, carry2):
                spl = plsc.load_gather(
                    alpha_v, [jnp.broadcast_to(p, (16,)),
                              jnp.broadcast_to(r, (16,))])
                for cc in range(DH // 16):
                    v = rows_v[p, r, pl.ds(cc * 16, 16)]
                    rows_v[p, r, pl.ds(cc * 16, 16)] = v * spl
                return carry2

            lax.fori_loop(0, BE, row_body, 0, unroll=4)

        def start_scatters(p):
            sc_acc(p).start(add=True)

            @pl.when(c == 0)
            def _():
                sc_den(p).start(add=True)

        def wait_scatters(p):
            sc_acc(p).wait()

            @pl.when(c == 0)
            def _():
                sc_den(p).wait()

        start_idx(0, 0)
        wait_idx(0, 0)
        scalar_phase(0)
        gather(0).start()
        start_idx(1, 1)

        def batch_body(b, carry):
            p = lax.bitwise_and(b, 1)
            q = 1 - p
            wait_idx(b, p)

            @pl.when(b >= 2)
            def _():
                wait_scatters(p)

            scalar_phase(p)
            gather(p).start()

            @pl.when(b < NBATCH - 1)
            def _():
                start_idx(b + 1, q)

            gather(q).wait()
            multiply(q)
            start_scatters(q)
            return carry

        lax.fori_loop(1, NBATCH, batch_body, 0)
        p_last = (NBATCH - 1) & 1
        gather(p_last).wait()
        multiply(p_last)
        start_scatters(p_last)
        wait_scatters(1 - p_last)
        wait_scatters(p_last)
        plsc.subcore_barrier()
        out0 = pl.multiple_of(c * NPAD + row0, 8)
        pltpu.sync_copy(acc_sh.at[pl.ds(row0, RPT)],
                        acc_out.at[pl.ds(out0, RPT)])

        @pl.when(c == 0)
        def _out_den():
            pltpu.sync_copy(den_sh.at[pl.ds(row0, RPT)],
                            den_out.at[pl.ds(row0, RPT)])

        @pl.when(s == NS - 1)
        def _out_tail():
            t0 = pl.multiple_of(c * NPAD + NS * RPT, 8)
            pltpu.sync_copy(acc_sh.at[pl.ds(NS * RPT, NTAIL)],
                            acc_out.at[pl.ds(t0, NTAIL)])

            @pl.when(c == 0)
            def _out_den_tail():
                pltpu.sync_copy(den_sh.at[pl.ds(NS * RPT, NTAIL)],
                                den_out.at[pl.ds(NS * RPT, NTAIL)])

    return k(hflat, aux, src, dst, ew, wa, z64, z16)


def _dot_t(a, b):
    return lax.dot_general(a, b, (((1,), (1,)), ((), ())),
                           preferred_element_type=jnp.float32)


def _split_cols(h, j):
    return jnp.where(j == 0, h[:, :DH], h[:, DH:])


def _tc_embed(x, W, asv, adv):
    def body(x_ref, w_ref, as_ref, ad_ref, h_ref, aux_ref, hs_ref):
        h = _dot_t(x_ref[...], w_ref[...])
        h_ref[...] = h
        aux_ref[...] = jnp.zeros((8, BN), jnp.float32)
        aux_ref[0:1, :] = _dot_t(as_ref[...], h)
        aux_ref[1:2, :] = _dot_t(ad_ref[...], h)
        hs_ref[...] = _split_cols(h, pl.program_id(1))

    return pl.pallas_call(
        body,
        grid=(NBLK, NC),
        in_specs=[
            pl.BlockSpec((BN, D), lambda i, j: (i, 0)),
            pl.BlockSpec((D, D), lambda i, j: (0, 0)),
            pl.BlockSpec((1, D), lambda i, j: (0, 0)),
            pl.BlockSpec((1, D), lambda i, j: (0, 0)),
        ],
        out_specs=[
            pl.BlockSpec((BN, D), lambda i, j: (i, 0)),
            pl.BlockSpec((8, BN), lambda i, j: (0, i)),
            pl.BlockSpec((BN, DH), lambda i, j: (j * NBLK + i, 0)),
        ],
        out_shape=[
            jax.ShapeDtypeStruct((N, D), jnp.float32),
            jax.ShapeDtypeStruct((8, N), jnp.float32),
            jax.ShapeDtypeStruct((NC * NPAD, DH), jnp.float32),
        ],
    )(x, W, asv, adv)


def _self_alpha(aux_ref, wa_ref, la):
    auxT = lax.dot_general(aux_ref[...], jnp.eye(8, dtype=jnp.float32),
                           (((0,), (0,)), ((), ())),
                           preferred_element_type=jnp.float32)
    wa = wa_ref[...]
    ce = jnp.sum(wa[0:1, :] * wa[1:2, :])
    logit = auxT[:, 0:1] + auxT[:, 1:2] + ce * la
    logit = jnp.where(logit > 0, logit, logit * jnp.float32(0.2))
    return jnp.exp(logit)


def _tc_combine1(acc, den, h1, aux1, wa1, b1, W2, as2v, ad2v):
    def body(alo_ref, ahi_ref, den_ref, h_ref, aux_ref, wa_ref, b_ref, w2_ref,
             as_ref, ad_ref, h2_ref, aux2_ref, la_ref, h2s_ref):
        dsum = den_ref[...]
        asum = dsum[:, 0:1]
        deg = dsum[:, 1:2]
        ews = dsum[:, 2:3]
        la = ews / jnp.maximum(deg, 1.0)
        aself = _self_alpha(aux_ref, wa_ref, la)
        dentot = asum + aself + jnp.float32(1e-16)
        accsum = jnp.concatenate([alo_ref[...], ahi_ref[...]], axis=-1)
        out1 = (accsum + aself * h_ref[...]) / dentot + b_ref[...]
        x2 = jnp.maximum(out1, 0.0)
        h2 = _dot_t(x2, w2_ref[...])
        h2_ref[...] = h2
        aux2_ref[...] = jnp.zeros((8, BN), jnp.float32)
        aux2_ref[0:1, :] = _dot_t(as_ref[...], h2)
        aux2_ref[1:2, :] = _dot_t(ad_ref[...], h2)
        la_ref[...] = la
        h2s_ref[...] = _split_cols(h2, pl.program_id(1))

    return pl.pallas_call(
        body,
        grid=(NBLK, NC),
        in_specs=[
            pl.BlockSpec((BN, DH), lambda i, j: (i, 0)),
            pl.BlockSpec((BN, DH), lambda i, j: (NBLK + i, 0)),
            pl.BlockSpec((BN, 16), lambda i, j: (i, 0)),
            pl.BlockSpec((BN, D), lambda i, j: (i, 0)),
            pl.BlockSpec((8, BN), lambda i, j: (0, i)),
            pl.BlockSpec((2, D), lambda i, j: (0, 0)),
            pl.BlockSpec((1, D), lambda i, j: (0, 0)),
            pl.BlockSpec((D, D), lambda i, j: (0, 0)),
            pl.BlockSpec((1, D), lambda i, j: (0, 0)),
            pl.BlockSpec((1, D), lambda i, j: (0, 0)),
        ],
        out_specs=[
            pl.BlockSpec((BN, D), lambda i, j: (i, 0)),
            pl.BlockSpec((8, BN), lambda i, j: (0, i)),
            pl.BlockSpec((BN, 1), lambda i, j: (i, 0)),
            pl.BlockSpec((BN, DH), lambda i, j: (j * NBLK + i, 0)),
        ],
        out_shape=[
            jax.ShapeDtypeStruct((N, D), jnp.float32),
            jax.ShapeDtypeStruct((8, N), jnp.float32),
            jax.ShapeDtypeStruct((N, 1), jnp.float32),
            jax.ShapeDtypeStruct((NC * NPAD, DH), jnp.float32),
        ],
    )(acc, acc, den, h1, aux1, wa1, b1, W2, as2v, ad2v)


def _tc_final(acc, den, h2, aux2, la, wa2, b2):
    def body(alo_ref, ahi_ref, den_ref, h_ref, aux_ref, la_ref, wa_ref, b_ref,
             out_ref):
        asum = den_ref[:, 0:1]
        la = la_ref[...]
        aself = _self_alpha(aux_ref, wa_ref, la)
        dentot = asum + aself + jnp.float32(1e-16)
        accsum = jnp.concatenate([alo_ref[...], ahi_ref[...]], axis=-1)
        out_ref[...] = (accsum + aself * h_ref[...]) / dentot + b_ref[...]

    return pl.pallas_call(
        body,
        grid=(NBLK,),
        in_specs=[
            pl.BlockSpec((BN, DH), lambda i: (i, 0)),
            pl.BlockSpec((BN, DH), lambda i: (NBLK + i, 0)),
            pl.BlockSpec((BN, 16), lambda i: (i, 0)),
            pl.BlockSpec((BN, D), lambda i: (i, 0)),
            pl.BlockSpec((8, BN), lambda i: (0, i)),
            pl.BlockSpec((BN, 1), lambda i: (i, 0)),
            pl.BlockSpec((2, D), lambda i: (0, 0)),
            pl.BlockSpec((1, D), lambda i: (0, 0)),
        ],
        out_specs=pl.BlockSpec((BN, D), lambda i: (i, 0)),
        out_shape=jax.ShapeDtypeStruct((N, D), jnp.float32),
    )(acc, acc, den, h2, aux2, la, wa2, b2)


def kernel(x, edge_index, edge_weight, W1, We1, as1, ad1, ae1, b1,
           W2, We2, as2, ad2, ae2, b2):
    src = edge_index[0]
    dst = edge_index[1]
    ew = edge_weight[:, 0]
    as1v = as1.reshape(1, D)
    ad1v = ad1.reshape(1, D)
    wa1 = jnp.concatenate([We1.reshape(1, D), ae1.reshape(1, D)], axis=0)
    as2v = as2.reshape(1, D)
    ad2v = ad2.reshape(1, D)
    wa2 = jnp.concatenate([We2.reshape(1, D), ae2.reshape(1, D)], axis=0)
    b1r = b1.reshape(1, D)
    b2r = b2.reshape(1, D)
    z64 = jnp.zeros((RPT, DH), jnp.float32)
    z16 = jnp.zeros((RPT, 16), jnp.float32)

    h1, aux1, h1s = _tc_embed(x, W1, as1v, ad1v)
    acc1, den1 = _sc_edge_pass(h1s, aux1, src, dst, ew, wa1, z64, z16)
    h2, aux2, la, h2s = _tc_combine1(acc1, den1, h1, aux1, wa1, b1r, W2,
                                     as2v, ad2v)
    acc2, den2 = _sc_edge_pass(h2s, aux2, src, dst, ew, wa2, z64, z16)
    return _tc_final(acc2, den2, h2, aux2, la, wa2, b2r)


# speedup vs baseline: 29.9198x; 1.0004x over previous
import functools

import jax
import jax.numpy as jnp
from jax import lax
from jax.experimental import pallas as pl
from jax.experimental.pallas import tpu as pltpu
from jax.experimental.pallas import tpu_sc as plsc

N = 10000
E = 320000
D = 128
DH = D // 2
NC = 2
NS = 16
EPT = E // NS
BE = 80
NBATCH = EPT // BE
RPT = 624
NTAIL = N - NS * RPT
BN = 512
NBLK = 20
NPAD = BN * NBLK


def _sc_edge_pass(hflat, aux, src, dst, ew, wa, z64, z16):
    mesh = plsc.VectorSubcoreMesh(core_axis_name="c", subcore_axis_name="s")

    @functools.partial(
        pl.kernel,
        out_type=(
            jax.ShapeDtypeStruct((NC * NPAD, DH), jnp.float32),
            jax.ShapeDtypeStruct((N, 16), jnp.float32),
        ),
        mesh=mesh,
        compiler_params=pltpu.CompilerParams(needs_layout_passes=False,
                                             use_tc_tiling_on_sc=False),
        scratch_types=[
            pltpu.VMEM_SHARED((N, DH), jnp.float32),
            pltpu.VMEM_SHARED((N, 16), jnp.float32),
            pltpu.VMEM((N,), jnp.float32),
            pltpu.VMEM((N,), jnp.float32),
            pltpu.VMEM((2, D), jnp.float32),
            pltpu.VMEM((2, BE), jnp.int32),
            pltpu.VMEM((2, BE), jnp.int32),
            pltpu.VMEM((2, BE), jnp.float32),
            pltpu.VMEM((2, BE), jnp.int32),
            pltpu.VMEM((2, BE), jnp.int32),
            pltpu.VMEM((2, BE), jnp.float32),
            pltpu.VMEM((2, BE, DH), jnp.float32),
            pltpu.VMEM((2, BE, 16), jnp.float32),
            pltpu.VMEM((16,), jnp.float32),
            pltpu.SemaphoreType.DMA,
            pltpu.SemaphoreType.DMA,
            pltpu.SemaphoreType.DMA,
            pltpu.SemaphoreType.DMA,
        ],
    )
    def k(h_hbm, aux_hbm, src_hbm, dst_hbm, ew_hbm, wa_hbm, z64_hbm, z16_hbm,
          acc_out, den_out,
          acc_sh, den_sh, asrc_t, adst_t, wa_t, srcb_v, dstb_v, ewb_v,
          gofs_v, dsti_v, alpha_v, rows_v, tails_v, red_v,
          sem_pk, sem_g, sem_sa, sem_sd):
        c = lax.axis_index("c")
        s = lax.axis_index("s")
        pltpu.sync_copy(aux_hbm.at[0], asrc_t)
        pltpu.sync_copy(aux_hbm.at[1], adst_t)
        pltpu.sync_copy(wa_hbm, wa_t)
        pltpu.sync_copy(z16_hbm.at[pl.ds(0, BE)], tails_v.at[0])
        pltpu.sync_copy(z16_hbm.at[pl.ds(0, BE)], tails_v.at[1])
        row0 = pl.multiple_of(s * RPT, 8)
        pltpu.sync_copy(z64_hbm, acc_sh.at[pl.ds(row0, RPT)])

        @pl.when(s == NS - 1)
        def _init_tail():
            pltpu.sync_copy(z64_hbm.at[pl.ds(0, NTAIL)],
                            acc_sh.at[pl.ds(NS * RPT, NTAIL)])

        @pl.when(c == 0)
        def _init_den():
            pltpu.sync_copy(z16_hbm, den_sh.at[pl.ds(row0, RPT)])

            @pl.when(s == NS - 1)
            def _init_den_tail():
                pltpu.sync_copy(z16_hbm.at[pl.ds(0, NTAIL)],
                                den_sh.at[pl.ds(NS * RPT, NTAIL)])

        lane16 = lax.iota(jnp.int32, 16)
        cev = jnp.zeros((16,), jnp.float32)
        for kk in range(D // 16):
            cev = cev + wa_t[0, pl.ds(kk * 16, 16)] * wa_t[1, pl.ds(kk * 16, 16)]
        for shift in (8, 4, 2, 1):
            red_v[...] = cev
            cev = cev + plsc.load_gather(red_v, [lane16 ^ shift])
        ce = cev
        col0 = jnp.zeros((16,), jnp.int32)
        col1 = col0 + 1
        col2 = col0 + 2
        ones16 = jnp.ones((16,), jnp.float32)
        gofs0 = c * NPAD
        plsc.subcore_barrier()
        ebase0 = s * EPT

        def ld_idx(b, p):
            eb = pl.multiple_of(ebase0 + b * BE, 8)
            return (pltpu.make_async_copy(src_hbm.at[pl.ds(eb, BE)],
                                          srcb_v.at[p], sem_pk),
                    pltpu.make_async_copy(dst_hbm.at[pl.ds(eb, BE)],
                                          dstb_v.at[p], sem_pk),
                    pltpu.make_async_copy(ew_hbm.at[pl.ds(eb, BE)],
                                          ewb_v.at[p], sem_pk))

        def start_idx(b, p):
            for d in ld_idx(b, p):
                d.start()

        def wait_idx(b, p):
            for d in ld_idx(b, p):
                d.wait()

        def gather(p):
            return pltpu.make_async_copy(h_hbm.at[gofs_v.at[p]],
                                         rows_v.at[p], sem_g)

        def sc_acc(p):
            return pltpu.make_async_copy(rows_v.at[p],
                                         acc_sh.at[dsti_v.at[p]], sem_sa)

        def sc_den(p):
            return pltpu.make_async_copy(tails_v.at[p],
                                         den_sh.at[dsti_v.at[p]], sem_sd)

        def scalar_phase(p):
            for g in range(BE // 16):
                off = g * 16
                si = srcb_v[p, pl.ds(off, 16)]
                di = dstb_v[p, pl.ds(off, 16)]
                ew16 = ewb_v[p, pl.ds(off, 16)]
                a = (plsc.load_gather(asrc_t, [si])
                     + plsc.load_gather(adst_t, [di]) + ce * ew16)
                a = jnp.where(a > 0, a, a * jnp.float32(0.2))
                a = jnp.exp(a)
                alpha_v[p, pl.ds(off, 16)] = a
                gofs_v[p, pl.ds(off, 16)] = si + gofs0
                dsti_v[p, pl.ds(off, 16)] = di
                bidx = jnp.broadcast_to(p, (16,))
                row16 = off + lane16
                plsc.store_scatter(tails_v, [bidx, row16, col0], a)
                plsc.store_scatter(tails_v, [bidx, row16, col1], ones16)
                plsc.store_scatter(tails_v, [bidx, row16, col2], ew16)

        def multiply(p):
            def row_body(r, carry2):
                spl = plsc.load_gather(
                    alpha_v, [jnp.broadcast_to(p, (16,)),
                              jnp.broadcast_to(r, (16,))])
                for cc in range(DH // 16):
                    v = rows_v[p, r, pl.ds(cc * 16, 16)]
                    rows_v[p, r, pl.ds(cc * 16, 16)] = v * spl
                return carry2

            lax.fori_loop(0, BE, row_body, 0, unroll=4)

        def start_scatters(p):
            sc_acc(p).start(add=True)

            @pl.when(c == 0)
            def _():
                sc_den(p).start(add=True)

        def wait_scatters(p):
            sc_acc(p).wait()

            @pl.when(c == 0)
            def _():
                sc_den(p).wait()

        start_idx(0, 0)
        wait_idx(0, 0)
        scalar_phase(0)
        gather(0).start()
        start_idx(1, 1)

        def batch_body(b, carry):
            p = lax.bitwise_and(b, 1)
            q = 1 - p
            wait_idx(b, p)

            @pl.when(b >= 2)
            def _():
                wait_scatters(p)

            scalar_phase(p)
            gather(p).start()

            @pl.when(b < NBATCH - 1)
            def _():
                start_idx(b + 1, q)

            gather(q).wait()
            multiply(q)
            start_scatters(q)
            return carry

        lax.fori_loop(1, NBATCH, batch_body, 0)
        p_last = (NBATCH - 1) & 1
        gather(p_last).wait()
        multiply(p_last)
        start_scatters(p_last)
        wait_scatters(1 - p_last)
        wait_scatters(p_last)
        plsc.subcore_barrier()
        out0 = pl.multiple_of(c * NPAD + row0, 8)
        pltpu.sync_copy(acc_sh.at[pl.ds(row0, RPT)],
                        acc_out.at[pl.ds(out0, RPT)])

        @pl.when(c == 0)
        def _out_den():
            pltpu.sync_copy(den_sh.at[pl.ds(row0, RPT)],
                            den_out.at[pl.ds(row0, RPT)])

        @pl.when(s == NS - 1)
        def _out_tail():
            t0 = pl.multiple_of(c * NPAD + NS * RPT, 8)
            pltpu.sync_copy(acc_sh.at[pl.ds(NS * RPT, NTAIL)],
                            acc_out.at[pl.ds(t0, NTAIL)])

            @pl.when(c == 0)
            def _out_den_tail():
                pltpu.sync_copy(den_sh.at[pl.ds(NS * RPT, NTAIL)],
                                den_out.at[pl.ds(NS * RPT, NTAIL)])

    return k(hflat, aux, src, dst, ew, wa, z64, z16)


def _dot_t(a, b):
    return lax.dot_general(a, b, (((1,), (1,)), ((), ())),
                           preferred_element_type=jnp.float32)


def _split_cols(h, j):
    return jnp.where(j == 0, h[:, :DH], h[:, DH:])


def _tc_embed(x, W, asv, adv):
    def body(x_ref, w_ref, as_ref, ad_ref, h_ref, aux_ref, hs_ref):
        h = _dot_t(x_ref[...], w_ref[...])
        h_ref[...] = h
        aux_ref[...] = jnp.zeros((8, BN), jnp.float32)
        aux_ref[0:1, :] = _dot_t(as_ref[...], h)
        aux_ref[1:2, :] = _dot_t(ad_ref[...], h)
        hs_ref[...] = _split_cols(h, pl.program_id(1))

    return pl.pallas_call(
        body,
        grid=(NBLK, NC),
        in_specs=[
            pl.BlockSpec((BN, D), lambda i, j: (i, 0)),
            pl.BlockSpec((D, D), lambda i, j: (0, 0)),
            pl.BlockSpec((1, D), lambda i, j: (0, 0)),
            pl.BlockSpec((1, D), lambda i, j: (0, 0)),
        ],
        out_specs=[
            pl.BlockSpec((BN, D), lambda i, j: (i, 0)),
            pl.BlockSpec((8, BN), lambda i, j: (0, i)),
            pl.BlockSpec((BN, DH), lambda i, j: (j * NBLK + i, 0)),
        ],
        out_shape=[
            jax.ShapeDtypeStruct((N, D), jnp.float32),
            jax.ShapeDtypeStruct((8, N), jnp.float32),
            jax.ShapeDtypeStruct((NC * NPAD, DH), jnp.float32),
        ],
    )(x, W, asv, adv)


def _self_alpha(aux_ref, wa_ref, la):
    auxT = lax.dot_general(aux_ref[...], jnp.eye(8, dtype=jnp.float32),
                           (((0,), (0,)), ((), ())),
                           preferred_element_type=jnp.float32)
    wa = wa_ref[...]
    ce = jnp.sum(wa[0:1, :] * wa[1:2, :])
    logit = auxT[:, 0:1] + auxT[:, 1:2] + ce * la
    logit = jnp.where(logit > 0, logit, logit * jnp.float32(0.2))
    return jnp.exp(logit)


def _tc_combine1(acc, den, h1, aux1, wa1, b1, W2, as2v, ad2v):
    def body(alo_ref, ahi_ref, den_ref, h_ref, aux_ref, wa_ref, b_ref, w2_ref,
             as_ref, ad_ref, h2_ref, aux2_ref, la_ref, h2s_ref):
        dsum = den_ref[...]
        asum = dsum[:, 0:1]
        deg = dsum[:, 1:2]
        ews = dsum[:, 2:3]
        la = ews / jnp.maximum(deg, 1.0)
        aself = _self_alpha(aux_ref, wa_ref, la)
        dentot = asum + aself + jnp.float32(1e-16)
        accsum = jnp.concatenate([alo_ref[...], ahi_ref[...]], axis=-1)
        out1 = (accsum + aself * h_ref[...]) / dentot + b_ref[...]
        x2 = jnp.maximum(out1, 0.0)
        h2 = _dot_t(x2, w2_ref[...])
        h2_ref[...] = h2
        aux2_ref[...] = jnp.zeros((8, BN), jnp.float32)
        aux2_ref[0:1, :] = _dot_t(as_ref[...], h2)
        aux2_ref[1:2, :] = _dot_t(ad_ref[...], h2)
        la_ref[...] = la
        h2s_ref[...] = _split_cols(h2, pl.program_id(1))

    return pl.pallas_call(
        body,
        grid=(NBLK, NC),
        in_specs=[
            pl.BlockSpec((BN, DH), lambda i, j: (i, 0)),
            pl.BlockSpec((BN, DH), lambda i, j: (NBLK + i, 0)),
            pl.BlockSpec((BN, 16), lambda i, j: (i, 0)),
            pl.BlockSpec((BN, D), lambda i, j: (i, 0)),
            pl.BlockSpec((8, BN), lambda i, j: (0, i)),
            pl.BlockSpec((2, D), lambda i, j: (0, 0)),
            pl.BlockSpec((1, D), lambda i, j: (0, 0)),
            pl.BlockSpec((D, D), lambda i, j: (0, 0)),
            pl.BlockSpec((1, D), lambda i, j: (0, 0)),
            pl.BlockSpec((1, D), lambda i, j: (0, 0)),
        ],
        out_specs=[
            pl.BlockSpec((BN, D), lambda i, j: (i, 0)),
            pl.BlockSpec((8, BN), lambda i, j: (0, i)),
            pl.BlockSpec((BN, 1), lambda i, j: (i, 0)),
            pl.BlockSpec((BN, DH), lambda i, j: (j * NBLK + i, 0)),
        ],
        out_shape=[
            jax.ShapeDtypeStruct((N, D), jnp.float32),
            jax.ShapeDtypeStruct((8, N), jnp.float32),
            jax.ShapeDtypeStruct((N, 1), jnp.float32),
            jax.ShapeDtypeStruct((NC * NPAD, DH), jnp.float32),
        ],
    )(acc, acc, den, h1, aux1, wa1, b1, W2, as2v, ad2v)


def _tc_final(acc, den, h2, aux2, la, wa2, b2):
    def body(alo_ref, ahi_ref, den_ref, h_ref, aux_ref, la_ref, wa_ref, b_ref,
             out_ref):
        asum = den_ref[:, 0:1]
        la = la_ref[...]
        aself = _self_alpha(aux_ref, wa_ref, la)
        dentot = asum + aself + jnp.float32(1e-16)
        accsum = jnp.concatenate([alo_ref[...], ahi_ref[...]], axis=-1)
        out_ref[...] = (accsum + aself * h_ref[...]) / dentot + b_ref[...]

    return pl.pallas_call(
        body,
        grid=(NBLK,),
        in_specs=[
            pl.BlockSpec((BN, DH), lambda i: (i, 0)),
            pl.BlockSpec((BN, DH), lambda i: (NBLK + i, 0)),
            pl.BlockSpec((BN, 16), lambda i: (i, 0)),
            pl.BlockSpec((BN, D), lambda i: (i, 0)),
            pl.BlockSpec((8, BN), lambda i: (0, i)),
            pl.BlockSpec((BN, 1), lambda i: (i, 0)),
            pl.BlockSpec((2, D), lambda i: (0, 0)),
            pl.BlockSpec((1, D), lambda i: (0, 0)),
        ],
        out_specs=pl.BlockSpec((BN, D), lambda i: (i, 0)),
        out_shape=jax.ShapeDtypeStruct((N, D), jnp.float32),
    )(acc, acc, den, h2, aux2, la, wa2, b2)


def kernel(x, edge_index, edge_weight, W1, We1, as1, ad1, ae1, b1,
           W2, We2, as2, ad2, ae2, b2):
    src = edge_index[0]
    dst = edge_index[1]
    ew = edge_weight[:, 0]
    as1v = as1.reshape(1, D)
    ad1v = ad1.reshape(1, D)
    wa1 = jnp.concatenate([We1.reshape(1, D), ae1.reshape(1, D)], axis=0)
    as2v = as2.reshape(1, D)
    ad2v = ad2.reshape(1, D)
    wa2 = jnp.concatenate([We2.reshape(1, D), ae2.reshape(1, D)], axis=0)
    b1r = b1.reshape(1, D)
    b2r = b2.reshape(1, D)
    z64 = jnp.zeros((RPT, DH), jnp.float32)
    z16 = jnp.zeros((RPT, 16), jnp.float32)

    h1, aux1, h1s = _tc_embed(x, W1, as1v, ad1v)
    acc1, den1 = _sc_edge_pass(h1s, aux1, src, dst, ew, wa1, z64, z16)
    h2, aux2, la, h2s = _tc_combine1(acc1, den1, h1, aux1, wa1, b1r, W2,
                                     as2v, ad2v)
    acc2, den2 = _sc_edge_pass(h2s, aux2, src, dst, ew, wa2, z64, z16)
    return _tc_final(acc2, den2, h2, aux2, la, wa2, b2r)


# speedup vs baseline: 33.3671x; 1.1152x over previous
import functools

import jax
import jax.numpy as jnp
from jax import lax
from jax.experimental import pallas as pl
from jax.experimental.pallas import tpu as pltpu
from jax.experimental.pallas import tpu_sc as plsc

N = 10000
E = 320000
D = 128
DH = D // 2
NC = 2
NS = 16
EPT = E // NS
BE = 80
NBATCH = EPT // BE
RPT = 624
NTAIL = N - NS * RPT
BN = 512
NBLK = 20
NPAD = BN * NBLK


def _sc_edge_pass(hflat, aux, src, dst, ew, wa, z64, z16):
    mesh = plsc.VectorSubcoreMesh(core_axis_name="c", subcore_axis_name="s")

    @functools.partial(
        pl.kernel,
        out_type=(
            jax.ShapeDtypeStruct((NC * NPAD, DH), jnp.float32),
            jax.ShapeDtypeStruct((N, 16), jnp.float32),
        ),
        mesh=mesh,
        compiler_params=pltpu.CompilerParams(needs_layout_passes=False,
                                             use_tc_tiling_on_sc=False),
        scratch_types=[
            pltpu.VMEM_SHARED((N, DH), jnp.float32),
            pltpu.VMEM_SHARED((N, 16), jnp.float32),
            pltpu.VMEM((N,), jnp.float32),
            pltpu.VMEM((N,), jnp.float32),
            pltpu.VMEM((2, D), jnp.float32),
            pltpu.VMEM((2, BE), jnp.int32),
            pltpu.VMEM((2, BE), jnp.int32),
            pltpu.VMEM((2, BE), jnp.float32),
            pltpu.VMEM((2, BE), jnp.int32),
            pltpu.VMEM((2, BE), jnp.int32),
            pltpu.VMEM((2, BE), jnp.float32),
            pltpu.VMEM((2, BE, DH), jnp.float32),
            pltpu.VMEM((2, BE, 16), jnp.float32),
            pltpu.VMEM((16,), jnp.float32),
            pltpu.SemaphoreType.DMA,
            pltpu.SemaphoreType.DMA,
            pltpu.SemaphoreType.DMA,
            pltpu.SemaphoreType.DMA,
        ],
    )
    def k(h_hbm, aux_hbm, src_hbm, dst_hbm, ew_hbm, wa_hbm, z64_hbm, z16_hbm,
          acc_out, den_out,
          acc_sh, den_sh, asrc_t, adst_t, wa_t, srcb_v, dstb_v, ewb_v,
          gofs_v, dsti_v, alpha_v, rows_v, tails_v, red_v,
          sem_pk, sem_g, sem_sa, sem_sd):
        c = lax.axis_index("c")
        s = lax.axis_index("s")
        pltpu.sync_copy(aux_hbm.at[0], asrc_t)
        pltpu.sync_copy(aux_hbm.at[1], adst_t)
        pltpu.sync_copy(wa_hbm, wa_t)
        pltpu.sync_copy(z16_hbm.at[pl.ds(0, BE)], tails_v.at[0])
        pltpu.sync_copy(z16_hbm.at[pl.ds(0, BE)], tails_v.at[1])
        row0 = pl.multiple_of(s * RPT, 8)
        pltpu.sync_copy(z64_hbm, acc_sh.at[pl.ds(row0, RPT)])

        @pl.when(s == NS - 1)
        def _init_tail():
            pltpu.sync_copy(z64_hbm.at[pl.ds(0, NTAIL)],
                            acc_sh.at[pl.ds(NS * RPT, NTAIL)])

        @pl.when(c == 0)
        def _init_den():
            pltpu.sync_copy(z16_hbm, den_sh.at[pl.ds(row0, RPT)])

            @pl.when(s == NS - 1)
            def _init_den_tail():
                pltpu.sync_copy(z16_hbm.at[pl.ds(0, NTAIL)],
                                den_sh.at[pl.ds(NS * RPT, NTAIL)])

        lane16 = lax.iota(jnp.int32, 16)
        cev = jnp.zeros((16,), jnp.float32)
        for kk in range(D // 16):
            cev = cev + wa_t[0, pl.ds(kk * 16, 16)] * wa_t[1, pl.ds(kk * 16, 16)]
        for shift in (8, 4, 2, 1):
            red_v[...] = cev
            cev = cev + plsc.load_gather(red_v, [lane16 ^ shift])
        ce = cev
        col0 = jnp.zeros((16,), jnp.int32)
        col1 = col0 + 1
        col2 = col0 + 2
        ones16 = jnp.ones((16,), jnp.float32)
        gofs0 = c * NPAD
        plsc.subcore_barrier()
        ebase0 = s * EPT

        def ld_idx(b, p):
            eb = pl.multiple_of(ebase0 + b * BE, 8)
            return (pltpu.make_async_copy(src_hbm.at[pl.ds(eb, BE)],
                                          srcb_v.at[p], sem_pk),
                    pltpu.make_async_copy(dst_hbm.at[pl.ds(eb, BE)],
                                          dstb_v.at[p], sem_pk),
                    pltpu.make_async_copy(ew_hbm.at[pl.ds(eb, BE)],
                                          ewb_v.at[p], sem_pk))

        def start_idx(b, p):
            for d in ld_idx(b, p):
                d.start()

        def wait_idx(b, p):
            for d in ld_idx(b, p):
                d.wait()

        def gather(p):
            return pltpu.make_async_copy(h_hbm.at[gofs_v.at[p]],
                                         rows_v.at[p], sem_g)

        def sc_acc(p):
            return pltpu.make_async_copy(rows_v.at[p],
                                         acc_sh.at[dsti_v.at[p]], sem_sa)

        def sc_den(p):
            return pltpu.make_async_copy(tails_v.at[p],
                                         den_sh.at[dsti_v.at[p]], sem_sd)

        def scalar_phase(p):
            for g in range(BE // 16):
                off = g * 16
                si = srcb_v[p, pl.ds(off, 16)]
                di = dstb_v[p, pl.ds(off, 16)]
                ew16 = ewb_v[p, pl.ds(off, 16)]
                a = (plsc.load_gather(asrc_t, [si])
                     + plsc.load_gather(adst_t, [di]) + ce * ew16)
                a = jnp.where(a > 0, a, a * jnp.float32(0.2))
                a = jnp.exp(a)
                alpha_v[p, pl.ds(off, 16)] = a
                gofs_v[p, pl.ds(off, 16)] = si + gofs0
                dsti_v[p, pl.ds(off, 16)] = di
                bidx = jnp.broadcast_to(p, (16,))
                row16 = off + lane16
                plsc.store_scatter(tails_v, [bidx, row16, col0], a)
                plsc.store_scatter(tails_v, [bidx, row16, col1], ones16)
                plsc.store_scatter(tails_v, [bidx, row16, col2], ew16)

        def multiply(p):
            def grp_body(g, carry2):
                av = alpha_v[p, pl.ds(pl.multiple_of(g * 16, 16), 16)]
                for j in range(16):
                    r = g * 16 + j
                    spl = lax.gather(
                        av, (col0 + j)[:, None],
                        lax.GatherDimensionNumbers(
                            offset_dims=(), collapsed_slice_dims=(0,),
                            start_index_map=(0,)),
                        (1,), mode=lax.GatherScatterMode.PROMISE_IN_BOUNDS)
                    for cc in range(DH // 16):
                        v = rows_v[p, r, pl.ds(cc * 16, 16)]
                        rows_v[p, r, pl.ds(cc * 16, 16)] = v * spl
                return carry2

            lax.fori_loop(0, BE // 16, grp_body, 0, unroll=1)

        def start_scatters(p):
            sc_acc(p).start(add=True)

            @pl.when(c == 0)
            def _():
                sc_den(p).start(add=True)

        def wait_scatters(p):
            sc_acc(p).wait()

            @pl.when(c == 0)
            def _():
                sc_den(p).wait()

        start_idx(0, 0)
        wait_idx(0, 0)
        scalar_phase(0)
        gather(0).start()
        start_idx(1, 1)

        def batch_body(b, carry):
            p = lax.bitwise_and(b, 1)
            q = 1 - p
            wait_idx(b, p)

            @pl.when(b >= 2)
            def _():
                wait_scatters(p)

            scalar_phase(p)
            gather(p).start()

            @pl.when(b < NBATCH - 1)
            def _():
                start_idx(b + 1, q)

            gather(q).wait()
            multiply(q)
            start_scatters(q)
            return carry

        lax.fori_loop(1, NBATCH, batch_body, 0)
        p_last = (NBATCH - 1) & 1
        gather(p_last).wait()
        multiply(p_last)
        start_scatters(p_last)
        wait_scatters(1 - p_last)
        wait_scatters(p_last)
        plsc.subcore_barrier()
        out0 = pl.multiple_of(c * NPAD + row0, 8)
        pltpu.sync_copy(acc_sh.at[pl.ds(row0, RPT)],
                        acc_out.at[pl.ds(out0, RPT)])

        @pl.when(c == 0)
        def _out_den():
            pltpu.sync_copy(den_sh.at[pl.ds(row0, RPT)],
                            den_out.at[pl.ds(row0, RPT)])

        @pl.when(s == NS - 1)
        def _out_tail():
            t0 = pl.multiple_of(c * NPAD + NS * RPT, 8)
            pltpu.sync_copy(acc_sh.at[pl.ds(NS * RPT, NTAIL)],
                            acc_out.at[pl.ds(t0, NTAIL)])

            @pl.when(c == 0)
            def _out_den_tail():
                pltpu.sync_copy(den_sh.at[pl.ds(NS * RPT, NTAIL)],
                                den_out.at[pl.ds(NS * RPT, NTAIL)])

    return k(hflat, aux, src, dst, ew, wa, z64, z16)


def _dot_t(a, b):
    return lax.dot_general(a, b, (((1,), (1,)), ((), ())),
                           preferred_element_type=jnp.float32)


def _split_cols(h, j):
    return jnp.where(j == 0, h[:, :DH], h[:, DH:])


def _tc_embed(x, W, asv, adv):
    def body(x_ref, w_ref, as_ref, ad_ref, h_ref, aux_ref, hs_ref):
        h = _dot_t(x_ref[...], w_ref[...])
        h_ref[...] = h
        aux_ref[...] = jnp.zeros((8, BN), jnp.float32)
        aux_ref[0:1, :] = _dot_t(as_ref[...], h)
        aux_ref[1:2, :] = _dot_t(ad_ref[...], h)
        hs_ref[...] = _split_cols(h, pl.program_id(1))

    return pl.pallas_call(
        body,
        grid=(NBLK, NC),
        in_specs=[
            pl.BlockSpec((BN, D), lambda i, j: (i, 0)),
            pl.BlockSpec((D, D), lambda i, j: (0, 0)),
            pl.BlockSpec((1, D), lambda i, j: (0, 0)),
            pl.BlockSpec((1, D), lambda i, j: (0, 0)),
        ],
        out_specs=[
            pl.BlockSpec((BN, D), lambda i, j: (i, 0)),
            pl.BlockSpec((8, BN), lambda i, j: (0, i)),
            pl.BlockSpec((BN, DH), lambda i, j: (j * NBLK + i, 0)),
        ],
        out_shape=[
            jax.ShapeDtypeStruct((N, D), jnp.float32),
            jax.ShapeDtypeStruct((8, N), jnp.float32),
            jax.ShapeDtypeStruct((NC * NPAD, DH), jnp.float32),
        ],
    )(x, W, asv, adv)


def _self_alpha(aux_ref, wa_ref, la):
    auxT = lax.dot_general(aux_ref[...], jnp.eye(8, dtype=jnp.float32),
                           (((0,), (0,)), ((), ())),
                           preferred_element_type=jnp.float32)
    wa = wa_ref[...]
    ce = jnp.sum(wa[0:1, :] * wa[1:2, :])
    logit = auxT[:, 0:1] + auxT[:, 1:2] + ce * la
    logit = jnp.where(logit > 0, logit, logit * jnp.float32(0.2))
    return jnp.exp(logit)


def _tc_combine1(acc, den, h1, aux1, wa1, b1, W2, as2v, ad2v):
    def body(alo_ref, ahi_ref, den_ref, h_ref, aux_ref, wa_ref, b_ref, w2_ref,
             as_ref, ad_ref, h2_ref, aux2_ref, la_ref, h2s_ref):
        dsum = den_ref[...]
        asum = dsum[:, 0:1]
        deg = dsum[:, 1:2]
        ews = dsum[:, 2:3]
        la = ews / jnp.maximum(deg, 1.0)
        aself = _self_alpha(aux_ref, wa_ref, la)
        dentot = asum + aself + jnp.float32(1e-16)
        accsum = jnp.concatenate([alo_ref[...], ahi_ref[...]], axis=-1)
        out1 = (accsum + aself * h_ref[...]) / dentot + b_ref[...]
        x2 = jnp.maximum(out1, 0.0)
        h2 = _dot_t(x2, w2_ref[...])
        h2_ref[...] = h2
        aux2_ref[...] = jnp.zeros((8, BN), jnp.float32)
        aux2_ref[0:1, :] = _dot_t(as_ref[...], h2)
        aux2_ref[1:2, :] = _dot_t(ad_ref[...], h2)
        la_ref[...] = la
        h2s_ref[...] = _split_cols(h2, pl.program_id(1))

    return pl.pallas_call(
        body,
        grid=(NBLK, NC),
        in_specs=[
            pl.BlockSpec((BN, DH), lambda i, j: (i, 0)),
            pl.BlockSpec((BN, DH), lambda i, j: (NBLK + i, 0)),
            pl.BlockSpec((BN, 16), lambda i, j: (i, 0)),
            pl.BlockSpec((BN, D), lambda i, j: (i, 0)),
            pl.BlockSpec((8, BN), lambda i, j: (0, i)),
            pl.BlockSpec((2, D), lambda i, j: (0, 0)),
            pl.BlockSpec((1, D), lambda i, j: (0, 0)),
            pl.BlockSpec((D, D), lambda i, j: (0, 0)),
            pl.BlockSpec((1, D), lambda i, j: (0, 0)),
            pl.BlockSpec((1, D), lambda i, j: (0, 0)),
        ],
        out_specs=[
            pl.BlockSpec((BN, D), lambda i, j: (i, 0)),
            pl.BlockSpec((8, BN), lambda i, j: (0, i)),
            pl.BlockSpec((BN, 1), lambda i, j: (i, 0)),
            pl.BlockSpec((BN, DH), lambda i, j: (j * NBLK + i, 0)),
        ],
        out_shape=[
            jax.ShapeDtypeStruct((N, D), jnp.float32),
            jax.ShapeDtypeStruct((8, N), jnp.float32),
            jax.ShapeDtypeStruct((N, 1), jnp.float32),
            jax.ShapeDtypeStruct((NC * NPAD, DH), jnp.float32),
        ],
    )(acc, acc, den, h1, aux1, wa1, b1, W2, as2v, ad2v)


def _tc_final(acc, den, h2, aux2, la, wa2, b2):
    def body(alo_ref, ahi_ref, den_ref, h_ref, aux_ref, la_ref, wa_ref, b_ref,
             out_ref):
        asum = den_ref[:, 0:1]
        la = la_ref[...]
        aself = _self_alpha(aux_ref, wa_ref, la)
        dentot = asum + aself + jnp.float32(1e-16)
        accsum = jnp.concatenate([alo_ref[...], ahi_ref[...]], axis=-1)
        out_ref[...] = (accsum + aself * h_ref[...]) / dentot + b_ref[...]

    return pl.pallas_call(
        body,
        grid=(NBLK,),
        in_specs=[
            pl.BlockSpec((BN, DH), lambda i: (i, 0)),
            pl.BlockSpec((BN, DH), lambda i: (NBLK + i, 0)),
            pl.BlockSpec((BN, 16), lambda i: (i, 0)),
            pl.BlockSpec((BN, D), lambda i: (i, 0)),
            pl.BlockSpec((8, BN), lambda i: (0, i)),
            pl.BlockSpec((BN, 1), lambda i: (i, 0)),
            pl.BlockSpec((2, D), lambda i: (0, 0)),
            pl.BlockSpec((1, D), lambda i: (0, 0)),
        ],
        out_specs=pl.BlockSpec((BN, D), lambda i: (i, 0)),
        out_shape=jax.ShapeDtypeStruct((N, D), jnp.float32),
    )(acc, acc, den, h2, aux2, la, wa2, b2)


def kernel(x, edge_index, edge_weight, W1, We1, as1, ad1, ae1, b1,
           W2, We2, as2, ad2, ae2, b2):
    src = edge_index[0]
    dst = edge_index[1]
    ew = edge_weight[:, 0]
    as1v = as1.reshape(1, D)
    ad1v = ad1.reshape(1, D)
    wa1 = jnp.concatenate([We1.reshape(1, D), ae1.reshape(1, D)], axis=0)
    as2v = as2.reshape(1, D)
    ad2v = ad2.reshape(1, D)
    wa2 = jnp.concatenate([We2.reshape(1, D), ae2.reshape(1, D)], axis=0)
    b1r = b1.reshape(1, D)
    b2r = b2.reshape(1, D)
    z64 = jnp.zeros((RPT, DH), jnp.float32)
    z16 = jnp.zeros((RPT, 16), jnp.float32)

    h1, aux1, h1s = _tc_embed(x, W1, as1v, ad1v)
    acc1, den1 = _sc_edge_pass(h1s, aux1, src, dst, ew, wa1, z64, z16)
    h2, aux2, la, h2s = _tc_combine1(acc1, den1, h1, aux1, wa1, b1r, W2,
                                     as2v, ad2v)
    acc2, den2 = _sc_edge_pass(h2s, aux2, src, dst, ew, wa2, z64, z16)
    return _tc_final(acc2, den2, h2, aux2, la, wa2, b2r)


# speedup vs baseline: 35.6168x; 1.0674x over previous
import functools

import jax
import jax.numpy as jnp
from jax import lax
from jax.experimental import pallas as pl
from jax.experimental.pallas import tpu as pltpu
from jax.experimental.pallas import tpu_sc as plsc

N = 10000
E = 320000
D = 128
DH = D // 2
NC = 2
NS = 16
EPT = E // NS
BE = 80
NBATCH = EPT // BE
RPT = 624
NTAIL = N - NS * RPT
BN = 512
NBLK = 20
NPAD = BN * NBLK


def _sc_edge_pass(hflat, aux, src, dst, ew, wa, z64, z16):
    mesh = plsc.VectorSubcoreMesh(core_axis_name="c", subcore_axis_name="s")

    @functools.partial(
        pl.kernel,
        out_type=(
            jax.ShapeDtypeStruct((NC * NPAD, DH), jnp.float32),
            jax.ShapeDtypeStruct((N, 16), jnp.float32),
        ),
        mesh=mesh,
        compiler_params=pltpu.CompilerParams(needs_layout_passes=False,
                                             use_tc_tiling_on_sc=False),
        scratch_types=[
            pltpu.VMEM_SHARED((N, DH), jnp.float32),
            pltpu.VMEM_SHARED((N, 16), jnp.float32),
            pltpu.VMEM((N,), jnp.float32),
            pltpu.VMEM((N,), jnp.float32),
            pltpu.VMEM((2, D), jnp.float32),
            pltpu.VMEM((2, BE), jnp.int32),
            pltpu.VMEM((2, BE), jnp.int32),
            pltpu.VMEM((2, BE), jnp.float32),
            pltpu.VMEM((2, BE), jnp.int32),
            pltpu.VMEM((2, BE), jnp.int32),
            pltpu.VMEM((2, BE), jnp.float32),
            pltpu.VMEM((2, BE, DH), jnp.float32),
            pltpu.VMEM((2, BE, 16), jnp.float32),
            pltpu.VMEM((16,), jnp.float32),
            pltpu.SemaphoreType.DMA,
            pltpu.SemaphoreType.DMA,
            pltpu.SemaphoreType.DMA,
            pltpu.SemaphoreType.DMA,
        ],
    )
    def k(h_hbm, aux_hbm, src_hbm, dst_hbm, ew_hbm, wa_hbm, z64_hbm, z16_hbm,
          acc_out, den_out,
          acc_sh, den_sh, asrc_t, adst_t, wa_t, srcb_v, dstb_v, ewb_v,
          gofs_v, dsti_v, alpha_v, rows_v, tails_v, red_v,
          sem_pk, sem_g, sem_sa, sem_sd):
        c = lax.axis_index("c")
        s = lax.axis_index("s")
        pltpu.sync_copy(aux_hbm.at[0], asrc_t)
        pltpu.sync_copy(aux_hbm.at[1], adst_t)
        pltpu.sync_copy(wa_hbm, wa_t)
        pltpu.sync_copy(z16_hbm.at[pl.ds(0, BE)], tails_v.at[0])
        pltpu.sync_copy(z16_hbm.at[pl.ds(0, BE)], tails_v.at[1])
        row0 = pl.multiple_of(s * RPT, 8)
        pltpu.sync_copy(z64_hbm, acc_sh.at[pl.ds(row0, RPT)])

        @pl.when(s == NS - 1)
        def _init_tail():
            pltpu.sync_copy(z64_hbm.at[pl.ds(0, NTAIL)],
                            acc_sh.at[pl.ds(NS * RPT, NTAIL)])

        @pl.when(c == 0)
        def _init_den():
            pltpu.sync_copy(z16_hbm, den_sh.at[pl.ds(row0, RPT)])

            @pl.when(s == NS - 1)
            def _init_den_tail():
                pltpu.sync_copy(z16_hbm.at[pl.ds(0, NTAIL)],
                                den_sh.at[pl.ds(NS * RPT, NTAIL)])

        lane16 = lax.iota(jnp.int32, 16)
        cev = jnp.zeros((16,), jnp.float32)
        for kk in range(D // 16):
            cev = cev + wa_t[0, pl.ds(kk * 16, 16)] * wa_t[1, pl.ds(kk * 16, 16)]
        for shift in (8, 4, 2, 1):
            red_v[...] = cev
            cev = cev + plsc.load_gather(red_v, [lane16 ^ shift])
        ce = cev
        col0 = jnp.zeros((16,), jnp.int32)
        col1 = col0 + 1
        col2 = col0 + 2
        ones16 = jnp.ones((16,), jnp.float32)
        gofs0 = c * NPAD
        plsc.subcore_barrier()
        ebase0 = s * EPT

        def ld_idx(b, p):
            eb = pl.multiple_of(ebase0 + b * BE, 8)
            return (pltpu.make_async_copy(src_hbm.at[pl.ds(eb, BE)],
                                          srcb_v.at[p], sem_pk),
                    pltpu.make_async_copy(dst_hbm.at[pl.ds(eb, BE)],
                                          dstb_v.at[p], sem_pk),
                    pltpu.make_async_copy(ew_hbm.at[pl.ds(eb, BE)],
                                          ewb_v.at[p], sem_pk))

        def start_idx(b, p):
            for d in ld_idx(b, p):
                d.start()

        def wait_idx(b, p):
            for d in ld_idx(b, p):
                d.wait()

        def gather(p):
            return pltpu.make_async_copy(h_hbm.at[gofs_v.at[p]],
                                         rows_v.at[p], sem_g)

        def sc_acc(p):
            return pltpu.make_async_copy(rows_v.at[p],
                                         acc_sh.at[dsti_v.at[p]], sem_sa)

        def sc_den(p):
            return pltpu.make_async_copy(tails_v.at[p],
                                         den_sh.at[dsti_v.at[p]], sem_sd)

        def scalar_phase(p):
            for g in range(BE // 16):
                off = g * 16
                si = srcb_v[p, pl.ds(off, 16)]
                di = dstb_v[p, pl.ds(off, 16)]
                ew16 = ewb_v[p, pl.ds(off, 16)]
                a = (plsc.load_gather(asrc_t, [si])
                     + plsc.load_gather(adst_t, [di]) + ce * ew16)
                a = jnp.where(a > 0, a, a * jnp.float32(0.2))
                a = jnp.exp(a)
                alpha_v[p, pl.ds(off, 16)] = a
                gofs_v[p, pl.ds(off, 16)] = si + gofs0
                dsti_v[p, pl.ds(off, 16)] = di
                bidx = jnp.broadcast_to(p, (16,))
                row16 = off + lane16
                plsc.store_scatter(tails_v, [bidx, row16, col0], a)
                plsc.store_scatter(tails_v, [bidx, row16, col1], ones16)
                plsc.store_scatter(tails_v, [bidx, row16, col2], ew16)

        def multiply(p):
            def grp_body(g, carry2):
                av = alpha_v[p, pl.ds(pl.multiple_of(g * 16, 16), 16)]
                for j in range(16):
                    r = g * 16 + j
                    spl = lax.gather(
                        av, (col0 + j)[:, None],
                        lax.GatherDimensionNumbers(
                            offset_dims=(), collapsed_slice_dims=(0,),
                            start_index_map=(0,)),
                        (1,), mode=lax.GatherScatterMode.PROMISE_IN_BOUNDS)
                    for cc in range(DH // 16):
                        v = rows_v[p, r, pl.ds(cc * 16, 16)]
                        rows_v[p, r, pl.ds(cc * 16, 16)] = v * spl
                return carry2

            lax.fori_loop(0, BE // 16, grp_body, 0, unroll=5)

        def start_scatters(p):
            sc_acc(p).start(add=True)

            @pl.when(c == 0)
            def _():
                sc_den(p).start(add=True)

        def wait_scatters(p):
            sc_acc(p).wait()

            @pl.when(c == 0)
            def _():
                sc_den(p).wait()

        start_idx(0, 0)
        wait_idx(0, 0)
        scalar_phase(0)
        gather(0).start()
        start_idx(1, 1)

        def batch_body(b, carry):
            p = lax.bitwise_and(b, 1)
            q = 1 - p
            wait_idx(b, p)

            @pl.when(b >= 2)
            def _():
                wait_scatters(p)

            scalar_phase(p)
            gather(p).start()

            @pl.when(b < NBATCH - 1)
            def _():
                start_idx(b + 1, q)

            gather(q).wait()
            multiply(q)
            start_scatters(q)
            return carry

        lax.fori_loop(1, NBATCH, batch_body, 0)
        p_last = (NBATCH - 1) & 1
        gather(p_last).wait()
        multiply(p_last)
        start_scatters(p_last)
        wait_scatters(1 - p_last)
        wait_scatters(p_last)
        plsc.subcore_barrier()
        out0 = pl.multiple_of(c * NPAD + row0, 8)
        pltpu.sync_copy(acc_sh.at[pl.ds(row0, RPT)],
                        acc_out.at[pl.ds(out0, RPT)])

        @pl.when(c == 0)
        def _out_den():
            pltpu.sync_copy(den_sh.at[pl.ds(row0, RPT)],
                            den_out.at[pl.ds(row0, RPT)])

        @pl.when(s == NS - 1)
        def _out_tail():
            t0 = pl.multiple_of(c * NPAD + NS * RPT, 8)
            pltpu.sync_copy(acc_sh.at[pl.ds(NS * RPT, NTAIL)],
                            acc_out.at[pl.ds(t0, NTAIL)])

            @pl.when(c == 0)
            def _out_den_tail():
                pltpu.sync_copy(den_sh.at[pl.ds(NS * RPT, NTAIL)],
                                den_out.at[pl.ds(NS * RPT, NTAIL)])

    return k(hflat, aux, src, dst, ew, wa, z64, z16)


def _dot_t(a, b):
    return lax.dot_general(a, b, (((1,), (1,)), ((), ())),
                           preferred_element_type=jnp.float32)


def _split_cols(h, j):
    return jnp.where(j == 0, h[:, :DH], h[:, DH:])


def _tc_embed(x, W, asv, adv):
    def body(x_ref, w_ref, as_ref, ad_ref, h_ref, aux_ref, hs_ref):
        h = _dot_t(x_ref[...], w_ref[...])
        h_ref[...] = h
        aux_ref[...] = jnp.zeros((8, BN), jnp.float32)
        aux_ref[0:1, :] = _dot_t(as_ref[...], h)
        aux_ref[1:2, :] = _dot_t(ad_ref[...], h)
        hs_ref[...] = _split_cols(h, pl.program_id(1))

    return pl.pallas_call(
        body,
        grid=(NBLK, NC),
        in_specs=[
            pl.BlockSpec((BN, D), lambda i, j: (i, 0)),
            pl.BlockSpec((D, D), lambda i, j: (0, 0)),
            pl.BlockSpec((1, D), lambda i, j: (0, 0)),
            pl.BlockSpec((1, D), lambda i, j: (0, 0)),
        ],
        out_specs=[
            pl.BlockSpec((BN, D), lambda i, j: (i, 0)),
            pl.BlockSpec((8, BN), lambda i, j: (0, i)),
            pl.BlockSpec((BN, DH), lambda i, j: (j * NBLK + i, 0)),
        ],
        out_shape=[
            jax.ShapeDtypeStruct((N, D), jnp.float32),
            jax.ShapeDtypeStruct((8, N), jnp.float32),
            jax.ShapeDtypeStruct((NC * NPAD, DH), jnp.float32),
        ],
    )(x, W, asv, adv)


def _self_alpha(aux_ref, wa_ref, la):
    auxT = lax.dot_general(aux_ref[...], jnp.eye(8, dtype=jnp.float32),
                           (((0,), (0,)), ((), ())),
                           preferred_element_type=jnp.float32)
    wa = wa_ref[...]
    ce = jnp.sum(wa[0:1, :] * wa[1:2, :])
    logit = auxT[:, 0:1] + auxT[:, 1:2] + ce * la
    logit = jnp.where(logit > 0, logit, logit * jnp.float32(0.2))
    return jnp.exp(logit)


def _tc_combine1(acc, den, h1, aux1, wa1, b1, W2, as2v, ad2v):
    def body(alo_ref, ahi_ref, den_ref, h_ref, aux_ref, wa_ref, b_ref, w2_ref,
             as_ref, ad_ref, h2_ref, aux2_ref, la_ref, h2s_ref):
        dsum = den_ref[...]
        asum = dsum[:, 0:1]
        deg = dsum[:, 1:2]
        ews = dsum[:, 2:3]
        la = ews / jnp.maximum(deg, 1.0)
        aself = _self_alpha(aux_ref, wa_ref, la)
        dentot = asum + aself + jnp.float32(1e-16)
        rden = jnp.float32(1.0) / dentot
        h = h_ref[...]
        x2lo = jnp.maximum(
            (alo_ref[...] + aself * h[:, :DH]) * rden + b_ref[:, :DH], 0.0)
        x2hi = jnp.maximum(
            (ahi_ref[...] + aself * h[:, DH:]) * rden + b_ref[:, DH:], 0.0)
        h2 = _dot_t(x2lo, w2_ref[...][:, :DH]) + _dot_t(x2hi, w2_ref[...][:, DH:])
        h2_ref[...] = h2
        aux2_ref[...] = jnp.zeros((8, BN), jnp.float32)
        aux2_ref[0:1, :] = _dot_t(as_ref[...], h2)
        aux2_ref[1:2, :] = _dot_t(ad_ref[...], h2)
        la_ref[...] = la
        h2s_ref[...] = _split_cols(h2, pl.program_id(1))

    return pl.pallas_call(
        body,
        grid=(NBLK, NC),
        in_specs=[
            pl.BlockSpec((BN, DH), lambda i, j: (i, 0)),
            pl.BlockSpec((BN, DH), lambda i, j: (NBLK + i, 0)),
            pl.BlockSpec((BN, 16), lambda i, j: (i, 0)),
            pl.BlockSpec((BN, D), lambda i, j: (i, 0)),
            pl.BlockSpec((8, BN), lambda i, j: (0, i)),
            pl.BlockSpec((2, D), lambda i, j: (0, 0)),
            pl.BlockSpec((1, D), lambda i, j: (0, 0)),
            pl.BlockSpec((D, D), lambda i, j: (0, 0)),
            pl.BlockSpec((1, D), lambda i, j: (0, 0)),
            pl.BlockSpec((1, D), lambda i, j: (0, 0)),
        ],
        out_specs=[
            pl.BlockSpec((BN, D), lambda i, j: (i, 0)),
            pl.BlockSpec((8, BN), lambda i, j: (0, i)),
            pl.BlockSpec((BN, 1), lambda i, j: (i, 0)),
            pl.BlockSpec((BN, DH), lambda i, j: (j * NBLK + i, 0)),
        ],
        out_shape=[
            jax.ShapeDtypeStruct((N, D), jnp.float32),
            jax.ShapeDtypeStruct((8, N), jnp.float32),
            jax.ShapeDtypeStruct((N, 1), jnp.float32),
            jax.ShapeDtypeStruct((NC * NPAD, DH), jnp.float32),
        ],
    )(acc, acc, den, h1, aux1, wa1, b1, W2, as2v, ad2v)


def _tc_final(acc, den, h2, aux2, la, wa2, b2):
    def body(alo_ref, ahi_ref, den_ref, h_ref, aux_ref, la_ref, wa_ref, b_ref,
             out_ref):
        asum = den_ref[:, 0:1]
        la = la_ref[...]
        aself = _self_alpha(aux_ref, wa_ref, la)
        dentot = asum + aself + jnp.float32(1e-16)
        rden = jnp.float32(1.0) / dentot
        h = h_ref[...]
        out_ref[:, :DH] = (alo_ref[...] + aself * h[:, :DH]) * rden + b_ref[:, :DH]
        out_ref[:, DH:] = (ahi_ref[...] + aself * h[:, DH:]) * rden + b_ref[:, DH:]

    return pl.pallas_call(
        body,
        grid=(NBLK,),
        in_specs=[
            pl.BlockSpec((BN, DH), lambda i: (i, 0)),
            pl.BlockSpec((BN, DH), lambda i: (NBLK + i, 0)),
            pl.BlockSpec((BN, 16), lambda i: (i, 0)),
            pl.BlockSpec((BN, D), lambda i: (i, 0)),
            pl.BlockSpec((8, BN), lambda i: (0, i)),
            pl.BlockSpec((BN, 1), lambda i: (i, 0)),
            pl.BlockSpec((2, D), lambda i: (0, 0)),
            pl.BlockSpec((1, D), lambda i: (0, 0)),
        ],
        out_specs=pl.BlockSpec((BN, D), lambda i: (i, 0)),
        out_shape=jax.ShapeDtypeStruct((N, D), jnp.float32),
    )(acc, acc, den, h2, aux2, la, wa2, b2)


def kernel(x, edge_index, edge_weight, W1, We1, as1, ad1, ae1, b1,
           W2, We2, as2, ad2, ae2, b2):
    src = edge_index[0]
    dst = edge_index[1]
    ew = edge_weight[:, 0]
    as1v = as1.reshape(1, D)
    ad1v = ad1.reshape(1, D)
    wa1 = jnp.concatenate([We1.reshape(1, D), ae1.reshape(1, D)], axis=0)
    as2v = as2.reshape(1, D)
    ad2v = ad2.reshape(1, D)
    wa2 = jnp.concatenate([We2.reshape(1, D), ae2.reshape(1, D)], axis=0)
    b1r = b1.reshape(1, D)
    b2r = b2.reshape(1, D)
    z64 = jnp.zeros((RPT, DH), jnp.float32)
    z16 = jnp.zeros((RPT, 16), jnp.float32)

    h1, aux1, h1s = _tc_embed(x, W1, as1v, ad1v)
    acc1, den1 = _sc_edge_pass(h1s, aux1, src, dst, ew, wa1, z64, z16)
    h2, aux2, la, h2s = _tc_combine1(acc1, den1, h1, aux1, wa1, b1r, W2,
                                     as2v, ad2v)
    acc2, den2 = _sc_edge_pass(h2s, aux2, src, dst, ew, wa2, z64, z16)
    return _tc_final(acc2, den2, h2, aux2, la, wa2, b2r)


# speedup vs baseline: 38.4054x; 1.0783x over previous
import functools

import jax
import jax.numpy as jnp
from jax import lax
from jax.experimental import pallas as pl
from jax.experimental.pallas import tpu as pltpu
from jax.experimental.pallas import tpu_sc as plsc

N = 10000
E = 320000
D = 128
DH = D // 2
NC = 2
NS = 16
EPT = E // NS
BE = 80
NBATCH = EPT // BE
RPT = 624
NTAIL = N - NS * RPT
BN = 512
NBLK = 20
NPAD = BN * NBLK


def _sc_edge_pass(hflat, aux, ei, ew, wa, z64, z16):
    mesh = plsc.VectorSubcoreMesh(core_axis_name="c", subcore_axis_name="s")

    @functools.partial(
        pl.kernel,
        out_type=(
            jax.ShapeDtypeStruct((NC * NPAD, DH), jnp.float32),
            jax.ShapeDtypeStruct((N, 16), jnp.float32),
        ),
        mesh=mesh,
        compiler_params=pltpu.CompilerParams(needs_layout_passes=False,
                                             use_tc_tiling_on_sc=False),
        scratch_types=[
            pltpu.VMEM_SHARED((N, DH), jnp.float32),
            pltpu.VMEM_SHARED((N, 16), jnp.float32),
            pltpu.VMEM((N,), jnp.float32),
            pltpu.VMEM((N,), jnp.float32),
            pltpu.VMEM((2, D), jnp.float32),
            pltpu.VMEM((2, BE), jnp.int32),
            pltpu.VMEM((2, BE), jnp.int32),
            pltpu.VMEM((2, BE), jnp.float32),
            pltpu.VMEM((2, BE), jnp.int32),
            pltpu.VMEM((2, BE), jnp.int32),
            pltpu.VMEM((2, BE), jnp.float32),
            pltpu.VMEM((2, BE, DH), jnp.float32),
            pltpu.VMEM((2, BE, 16), jnp.float32),
            pltpu.VMEM((16,), jnp.float32),
            pltpu.SemaphoreType.DMA,
            pltpu.SemaphoreType.DMA,
            pltpu.SemaphoreType.DMA,
            pltpu.SemaphoreType.DMA,
        ],
    )
    def k(h_hbm, aux_hbm, ei_hbm, ew_hbm, wa_hbm, z64_hbm, z16_hbm,
          acc_out, den_out,
          acc_sh, den_sh, asrc_t, adst_t, wa_t, srcb_v, dstb_v, ewb_v,
          gofs_v, dsti_v, alpha_v, rows_v, tails_v, red_v,
          sem_pk, sem_g, sem_sa, sem_sd):
        c = lax.axis_index("c")
        s = lax.axis_index("s")
        pltpu.sync_copy(aux_hbm.at[0], asrc_t)
        pltpu.sync_copy(aux_hbm.at[1], adst_t)
        pltpu.sync_copy(wa_hbm, wa_t)
        pltpu.sync_copy(z16_hbm.at[pl.ds(0, BE)], tails_v.at[0])
        pltpu.sync_copy(z16_hbm.at[pl.ds(0, BE)], tails_v.at[1])
        row0 = pl.multiple_of(s * RPT, 8)
        pltpu.sync_copy(z64_hbm, acc_sh.at[pl.ds(row0, RPT)])

        @pl.when(s == NS - 1)
        def _init_tail():
            pltpu.sync_copy(z64_hbm.at[pl.ds(0, NTAIL)],
                            acc_sh.at[pl.ds(NS * RPT, NTAIL)])

        @pl.when(c == 0)
        def _init_den():
            pltpu.sync_copy(z16_hbm, den_sh.at[pl.ds(row0, RPT)])

            @pl.when(s == NS - 1)
            def _init_den_tail():
                pltpu.sync_copy(z16_hbm.at[pl.ds(0, NTAIL)],
                                den_sh.at[pl.ds(NS * RPT, NTAIL)])

        lane16 = lax.iota(jnp.int32, 16)
        cev = jnp.zeros((16,), jnp.float32)
        for kk in range(D // 16):
            cev = cev + wa_t[0, pl.ds(kk * 16, 16)] * wa_t[1, pl.ds(kk * 16, 16)]
        for shift in (8, 4, 2, 1):
            red_v[...] = cev
            cev = cev + plsc.load_gather(red_v, [lane16 ^ shift])
        ce = cev
        col0 = jnp.zeros((16,), jnp.int32)
        col1 = col0 + 1
        col2 = col0 + 2
        ones16 = jnp.ones((16,), jnp.float32)
        gofs0 = c * NPAD
        plsc.subcore_barrier()
        ebase0 = s * EPT

        def ld_idx(b, p):
            eb = pl.multiple_of(ebase0 + b * BE, 8)
            return (pltpu.make_async_copy(ei_hbm.at[0, pl.ds(eb, BE)],
                                          srcb_v.at[p], sem_pk),
                    pltpu.make_async_copy(ei_hbm.at[1, pl.ds(eb, BE)],
                                          dstb_v.at[p], sem_pk),
                    pltpu.make_async_copy(ew_hbm.at[pl.ds(eb, BE)],
                                          ewb_v.at[p], sem_pk))

        def start_idx(b, p):
            for d in ld_idx(b, p):
                d.start()

        def wait_idx(b, p):
            for d in ld_idx(b, p):
                d.wait()

        def gather(p):
            return pltpu.make_async_copy(h_hbm.at[gofs_v.at[p]],
                                         rows_v.at[p], sem_g)

        def sc_acc(p):
            return pltpu.make_async_copy(rows_v.at[p],
                                         acc_sh.at[dsti_v.at[p]], sem_sa)

        def sc_den(p):
            return pltpu.make_async_copy(tails_v.at[p],
                                         den_sh.at[dsti_v.at[p]], sem_sd)

        def scalar_phase(p):
            for g in range(BE // 16):
                off = g * 16
                si = srcb_v[p, pl.ds(off, 16)]
                di = dstb_v[p, pl.ds(off, 16)]
                ew16 = ewb_v[p, pl.ds(off, 16)]
                a = (plsc.load_gather(asrc_t, [si])
                     + plsc.load_gather(adst_t, [di]) + ce * ew16)
                a = jnp.where(a > 0, a, a * jnp.float32(0.2))
                a = jnp.exp(a)
                alpha_v[p, pl.ds(off, 16)] = a
                gofs_v[p, pl.ds(off, 16)] = si + gofs0
                dsti_v[p, pl.ds(off, 16)] = di
                bidx = jnp.broadcast_to(p, (16,))
                row16 = off + lane16
                plsc.store_scatter(tails_v, [bidx, row16, col0], a)
                plsc.store_scatter(tails_v, [bidx, row16, col1], ones16)
                plsc.store_scatter(tails_v, [bidx, row16, col2], ew16)

        def multiply(p):
            def grp_body(g, carry2):
                av = alpha_v[p, pl.ds(pl.multiple_of(g * 16, 16), 16)]
                for j in range(16):
                    r = g * 16 + j
                    spl = lax.gather(
                        av, (col0 + j)[:, None],
                        lax.GatherDimensionNumbers(
                            offset_dims=(), collapsed_slice_dims=(0,),
                            start_index_map=(0,)),
                        (1,), mode=lax.GatherScatterMode.PROMISE_IN_BOUNDS)
                    for cc in range(DH // 16):
                        v = rows_v[p, r, pl.ds(cc * 16, 16)]
                        rows_v[p, r, pl.ds(cc * 16, 16)] = v * spl
                return carry2

            lax.fori_loop(0, BE // 16, grp_body, 0, unroll=5)

        def start_scatters(p):
            sc_acc(p).start(add=True)

            @pl.when(c == 0)
            def _():
                sc_den(p).start(add=True)

        def wait_scatters(p):
            sc_acc(p).wait()

            @pl.when(c == 0)
            def _():
                sc_den(p).wait()

        start_idx(0, 0)
        wait_idx(0, 0)
        scalar_phase(0)
        gather(0).start()
        start_idx(1, 1)

        def batch_body(b, carry):
            p = lax.bitwise_and(b, 1)
            q = 1 - p
            wait_idx(b, p)

            @pl.when(b >= 2)
            def _():
                wait_scatters(p)

            scalar_phase(p)
            gather(p).start()

            @pl.when(b < NBATCH - 1)
            def _():
                start_idx(b + 1, q)

            gather(q).wait()
            multiply(q)
            start_scatters(q)
            return carry

        lax.fori_loop(1, NBATCH, batch_body, 0)
        p_last = (NBATCH - 1) & 1
        gather(p_last).wait()
        multiply(p_last)
        start_scatters(p_last)
        wait_scatters(1 - p_last)
        wait_scatters(p_last)
        plsc.subcore_barrier()
        out0 = pl.multiple_of(c * NPAD + row0, 8)
        pltpu.sync_copy(acc_sh.at[pl.ds(row0, RPT)],
                        acc_out.at[pl.ds(out0, RPT)])

        @pl.when(c == 0)
        def _out_den():
            pltpu.sync_copy(den_sh.at[pl.ds(row0, RPT)],
                            den_out.at[pl.ds(row0, RPT)])

        @pl.when(s == NS - 1)
        def _out_tail():
            t0 = pl.multiple_of(c * NPAD + NS * RPT, 8)
            pltpu.sync_copy(acc_sh.at[pl.ds(NS * RPT, NTAIL)],
                            acc_out.at[pl.ds(t0, NTAIL)])

            @pl.when(c == 0)
            def _out_den_tail():
                pltpu.sync_copy(den_sh.at[pl.ds(NS * RPT, NTAIL)],
                                den_out.at[pl.ds(NS * RPT, NTAIL)])

    return k(hflat, aux, ei, ew, wa, z64, z16)


def _dot_t(a, b):
    return lax.dot_general(a, b, (((1,), (1,)), ((), ())),
                           preferred_element_type=jnp.float32)


def _tc_embed(x, W, asv, adv):
    def body(x_ref, w_ref, as_ref, ad_ref, h_ref, aux_ref, hs_ref):
        h = _dot_t(x_ref[...], w_ref[...])
        h_ref[...] = h
        aux_ref[...] = jnp.zeros((8, BN), jnp.float32)
        aux_ref[0:1, :] = _dot_t(as_ref[...], h)
        aux_ref[1:2, :] = _dot_t(ad_ref[...], h)
        hs_ref[0] = h[:, :DH]
        hs_ref[1] = h[:, DH:]

    return pl.pallas_call(
        body,
        grid=(NBLK,),
        in_specs=[
            pl.BlockSpec((BN, D), lambda i: (i, 0)),
            pl.BlockSpec((D, D), lambda i: (0, 0)),
            pl.BlockSpec((1, D), lambda i: (0, 0)),
            pl.BlockSpec((1, D), lambda i: (0, 0)),
        ],
        out_specs=[
            pl.BlockSpec((BN, D), lambda i: (i, 0)),
            pl.BlockSpec((8, BN), lambda i: (0, i)),
            pl.BlockSpec((NC, BN, DH), lambda i: (0, i, 0)),
        ],
        out_shape=[
            jax.ShapeDtypeStruct((N, D), jnp.float32),
            jax.ShapeDtypeStruct((8, N), jnp.float32),
            jax.ShapeDtypeStruct((NC, NPAD, DH), jnp.float32),
        ],
    )(x, W, asv, adv)


def _self_alpha(aux_ref, wa_ref, la):
    auxT = lax.dot_general(aux_ref[...], jnp.eye(8, dtype=jnp.float32),
                           (((0,), (0,)), ((), ())),
                           preferred_element_type=jnp.float32)
    wa = wa_ref[...]
    ce = jnp.sum(wa[0:1, :] * wa[1:2, :])
    logit = auxT[:, 0:1] + auxT[:, 1:2] + ce * la
    logit = jnp.where(logit > 0, logit, logit * jnp.float32(0.2))
    return jnp.exp(logit)


def _tc_combine1(acc, den, h1, aux1, wa1, b1, W2, as2v, ad2v):
    def body(alo_ref, ahi_ref, den_ref, h_ref, aux_ref, wa_ref, b_ref, w2_ref,
             as_ref, ad_ref, h2_ref, aux2_ref, la_ref, h2s_ref):
        dsum = den_ref[...]
        asum = dsum[:, 0:1]
        deg = dsum[:, 1:2]
        ews = dsum[:, 2:3]
        la = ews / jnp.maximum(deg, 1.0)
        aself = _self_alpha(aux_ref, wa_ref, la)
        dentot = asum + aself + jnp.float32(1e-16)
        rden = jnp.float32(1.0) / dentot
        h = h_ref[...]
        x2lo = jnp.maximum(
            (alo_ref[0] + aself * h[:, :DH]) * rden + b_ref[:, :DH], 0.0)
        x2hi = jnp.maximum(
            (ahi_ref[0] + aself * h[:, DH:]) * rden + b_ref[:, DH:], 0.0)
        h2 = _dot_t(x2lo, w2_ref[...][:, :DH]) + _dot_t(x2hi, w2_ref[...][:, DH:])
        h2_ref[...] = h2
        aux2_ref[...] = jnp.zeros((8, BN), jnp.float32)
        aux2_ref[0:1, :] = _dot_t(as_ref[...], h2)
        aux2_ref[1:2, :] = _dot_t(ad_ref[...], h2)
        la_ref[...] = la
        h2s_ref[0] = h2[:, :DH]
        h2s_ref[1] = h2[:, DH:]

    return pl.pallas_call(
        body,
        grid=(NBLK,),
        in_specs=[
            pl.BlockSpec((1, BN, DH), lambda i: (0, i, 0)),
            pl.BlockSpec((1, BN, DH), lambda i: (1, i, 0)),
            pl.BlockSpec((BN, 16), lambda i: (i, 0)),
            pl.BlockSpec((BN, D), lambda i: (i, 0)),
            pl.BlockSpec((8, BN), lambda i: (0, i)),
            pl.BlockSpec((2, D), lambda i: (0, 0)),
            pl.BlockSpec((1, D), lambda i: (0, 0)),
            pl.BlockSpec((D, D), lambda i: (0, 0)),
            pl.BlockSpec((1, D), lambda i: (0, 0)),
            pl.BlockSpec((1, D), lambda i: (0, 0)),
        ],
        out_specs=[
            pl.BlockSpec((BN, D), lambda i: (i, 0)),
            pl.BlockSpec((8, BN), lambda i: (0, i)),
            pl.BlockSpec((BN, 1), lambda i: (i, 0)),
            pl.BlockSpec((NC, BN, DH), lambda i: (0, i, 0)),
        ],
        out_shape=[
            jax.ShapeDtypeStruct((N, D), jnp.float32),
            jax.ShapeDtypeStruct((8, N), jnp.float32),
            jax.ShapeDtypeStruct((N, 1), jnp.float32),
            jax.ShapeDtypeStruct((NC, NPAD, DH), jnp.float32),
        ],
    )(acc, acc, den, h1, aux1, wa1, b1, W2, as2v, ad2v)


def _tc_final(acc, den, h2, aux2, la, wa2, b2):
    def body(alo_ref, ahi_ref, den_ref, h_ref, aux_ref, la_ref, wa_ref, b_ref,
             out_ref):
        asum = den_ref[:, 0:1]
        la = la_ref[...]
        aself = _self_alpha(aux_ref, wa_ref, la)
        dentot = asum + aself + jnp.float32(1e-16)
        rden = jnp.float32(1.0) / dentot
        h = h_ref[...]
        out_ref[:, :DH] = (alo_ref[0] + aself * h[:, :DH]) * rden + b_ref[:, :DH]
        out_ref[:, DH:] = (ahi_ref[0] + aself * h[:, DH:]) * rden + b_ref[:, DH:]

    return pl.pallas_call(
        body,
        grid=(NBLK,),
        in_specs=[
            pl.BlockSpec((1, BN, DH), lambda i: (0, i, 0)),
            pl.BlockSpec((1, BN, DH), lambda i: (1, i, 0)),
            pl.BlockSpec((BN, 16), lambda i: (i, 0)),
            pl.BlockSpec((BN, D), lambda i: (i, 0)),
            pl.BlockSpec((8, BN), lambda i: (0, i)),
            pl.BlockSpec((BN, 1), lambda i: (i, 0)),
            pl.BlockSpec((2, D), lambda i: (0, 0)),
            pl.BlockSpec((1, D), lambda i: (0, 0)),
        ],
        out_specs=pl.BlockSpec((BN, D), lambda i: (i, 0)),
        out_shape=jax.ShapeDtypeStruct((N, D), jnp.float32),
    )(acc, acc, den, h2, aux2, la, wa2, b2)


def kernel(x, edge_index, edge_weight, W1, We1, as1, ad1, ae1, b1,
           W2, We2, as2, ad2, ae2, b2):
    ew = edge_weight.reshape(E)
    as1v = as1.reshape(1, D)
    ad1v = ad1.reshape(1, D)
    wa1 = jnp.concatenate([We1.reshape(1, D), ae1.reshape(1, D)], axis=0)
    as2v = as2.reshape(1, D)
    ad2v = ad2.reshape(1, D)
    wa2 = jnp.concatenate([We2.reshape(1, D), ae2.reshape(1, D)], axis=0)
    b1r = b1.reshape(1, D)
    b2r = b2.reshape(1, D)
    z64 = jnp.zeros((RPT, DH), jnp.float32)
    z16 = jnp.zeros((RPT, 16), jnp.float32)

    h1, aux1, h1s = _tc_embed(x, W1, as1v, ad1v)
    acc1, den1 = _sc_edge_pass(h1s.reshape(NC * NPAD, DH), aux1, edge_index,
                               ew, wa1, z64, z16)
    h2, aux2, la, h2s = _tc_combine1(acc1.reshape(NC, NPAD, DH), den1, h1,
                                     aux1, wa1, b1r, W2, as2v, ad2v)
    acc2, den2 = _sc_edge_pass(h2s.reshape(NC * NPAD, DH), aux2, edge_index,
                               ew, wa2, z64, z16)
    return _tc_final(acc2.reshape(NC, NPAD, DH), den2, h2, aux2, la, wa2, b2r)


# speedup vs baseline: 38.5967x; 1.0050x over previous
import functools

import jax
import jax.numpy as jnp
from jax import lax
from jax.experimental import pallas as pl
from jax.experimental.pallas import tpu as pltpu
from jax.experimental.pallas import tpu_sc as plsc

N = 10000
E = 320000
D = 128
DH = D // 2
NC = 2
NS = 16
EPT = E // NS
BE = 80
NBATCH = EPT // BE
RPT = 624
NTAIL = N - NS * RPT
BN = 512
NBLK = 20
NPAD = BN * NBLK


def _sc_edge_pass(hflat, aux, ei, ew, wa, z64, z16):
    mesh = plsc.VectorSubcoreMesh(core_axis_name="c", subcore_axis_name="s")

    @functools.partial(
        pl.kernel,
        out_type=(
            jax.ShapeDtypeStruct((NC * NPAD, DH), jnp.float32),
            jax.ShapeDtypeStruct((N, 16), jnp.float32),
        ),
        mesh=mesh,
        compiler_params=pltpu.CompilerParams(needs_layout_passes=False,
                                             use_tc_tiling_on_sc=False),
        scratch_types=[
            pltpu.VMEM_SHARED((N, DH), jnp.float32),
            pltpu.VMEM_SHARED((N, 16), jnp.float32),
            pltpu.VMEM((N,), jnp.float32),
            pltpu.VMEM((N,), jnp.float32),
            pltpu.VMEM((2, D), jnp.float32),
            pltpu.VMEM((2, BE), jnp.int32),
            pltpu.VMEM((2, BE), jnp.int32),
            pltpu.VMEM((2, BE), jnp.float32),
            pltpu.VMEM((2, BE), jnp.int32),
            pltpu.VMEM((2, BE), jnp.int32),
            pltpu.VMEM((2, BE), jnp.float32),
            pltpu.VMEM((2, BE, DH), jnp.float32),
            pltpu.VMEM((2, BE, 16), jnp.float32),
            pltpu.VMEM((16,), jnp.float32),
            pltpu.SemaphoreType.DMA,
            pltpu.SemaphoreType.DMA,
            pltpu.SemaphoreType.DMA,
            pltpu.SemaphoreType.DMA,
        ],
    )
    def k(h_hbm, aux_hbm, ei_hbm, ew_hbm, wa_hbm, z64_hbm, z16_hbm,
          acc_out, den_out,
          acc_sh, den_sh, asrc_t, adst_t, wa_t, srcb_v, dstb_v, ewb_v,
          gofs_v, dsti_v, alpha_v, rows_v, tails_v, red_v,
          sem_pk, sem_g, sem_sa, sem_sd):
        c = lax.axis_index("c")
        s = lax.axis_index("s")
        pltpu.sync_copy(aux_hbm.at[0], asrc_t)
        pltpu.sync_copy(aux_hbm.at[1], adst_t)
        pltpu.sync_copy(wa_hbm, wa_t)
        pltpu.sync_copy(z16_hbm.at[pl.ds(0, BE)], tails_v.at[0])
        pltpu.sync_copy(z16_hbm.at[pl.ds(0, BE)], tails_v.at[1])
        row0 = pl.multiple_of(s * RPT, 8)
        pltpu.sync_copy(z64_hbm, acc_sh.at[pl.ds(row0, RPT)])

        @pl.when(s == NS - 1)
        def _init_tail():
            pltpu.sync_copy(z64_hbm.at[pl.ds(0, NTAIL)],
                            acc_sh.at[pl.ds(NS * RPT, NTAIL)])

        @pl.when(c == 0)
        def _init_den():
            pltpu.sync_copy(z16_hbm, den_sh.at[pl.ds(row0, RPT)])

            @pl.when(s == NS - 1)
            def _init_den_tail():
                pltpu.sync_copy(z16_hbm.at[pl.ds(0, NTAIL)],
                                den_sh.at[pl.ds(NS * RPT, NTAIL)])

        lane16 = lax.iota(jnp.int32, 16)
        cev = jnp.zeros((16,), jnp.float32)
        for kk in range(D // 16):
            cev = cev + wa_t[0, pl.ds(kk * 16, 16)] * wa_t[1, pl.ds(kk * 16, 16)]
        for shift in (8, 4, 2, 1):
            red_v[...] = cev
            cev = cev + plsc.load_gather(red_v, [lane16 ^ shift])
        ce = cev
        col0 = jnp.zeros((16,), jnp.int32)
        col1 = col0 + 1
        col2 = col0 + 2
        ones16 = jnp.ones((16,), jnp.float32)
        gofs0 = c * NPAD
        plsc.subcore_barrier()
        ebase0 = s * EPT

        def ld_idx(b, p):
            eb = pl.multiple_of(ebase0 + b * BE, 8)
            return (pltpu.make_async_copy(ei_hbm.at[0, pl.ds(eb, BE)],
                                          srcb_v.at[p], sem_pk),
                    pltpu.make_async_copy(ei_hbm.at[1, pl.ds(eb, BE)],
                                          dstb_v.at[p], sem_pk),
                    pltpu.make_async_copy(ew_hbm.at[pl.ds(eb, BE)],
                                          ewb_v.at[p], sem_pk))

        def start_idx(b, p):
            for d in ld_idx(b, p):
                d.start()

        def wait_idx(b, p):
            for d in ld_idx(b, p):
                d.wait()

        def gather(p):
            return pltpu.make_async_copy(h_hbm.at[gofs_v.at[p]],
                                         rows_v.at[p], sem_g)

        def sc_acc(p):
            return pltpu.make_async_copy(rows_v.at[p],
                                         acc_sh.at[dsti_v.at[p]], sem_sa)

        def sc_den(p):
            return pltpu.make_async_copy(tails_v.at[p],
                                         den_sh.at[dsti_v.at[p]], sem_sd)

        def scalar_phase(p):
            for g in range(BE // 16):
                off = g * 16
                si = srcb_v[p, pl.ds(off, 16)]
                di = dstb_v[p, pl.ds(off, 16)]
                ew16 = ewb_v[p, pl.ds(off, 16)]
                a = (plsc.load_gather(asrc_t, [si])
                     + plsc.load_gather(adst_t, [di]) + ce * ew16)
                a = jnp.where(a > 0, a, a * jnp.float32(0.2))
                a = jnp.exp(a)
                alpha_v[p, pl.ds(off, 16)] = a
                gofs_v[p, pl.ds(off, 16)] = si + gofs0
                dsti_v[p, pl.ds(off, 16)] = di
                bidx = jnp.broadcast_to(p, (16,))
                row16 = off + lane16
                plsc.store_scatter(tails_v, [bidx, row16, col0], a)
                plsc.store_scatter(tails_v, [bidx, row16, col1], ones16)
                plsc.store_scatter(tails_v, [bidx, row16, col2], ew16)

        def multiply(p):
            def grp_body(g, carry2):
                av = alpha_v[p, pl.ds(pl.multiple_of(g * 16, 16), 16)]
                for j in range(16):
                    r = g * 16 + j
                    spl = lax.gather(
                        av, (col0 + j)[:, None],
                        lax.GatherDimensionNumbers(
                            offset_dims=(), collapsed_slice_dims=(0,),
                            start_index_map=(0,)),
                        (1,), mode=lax.GatherScatterMode.PROMISE_IN_BOUNDS)
                    for cc in range(DH // 16):
                        v = rows_v[p, r, pl.ds(cc * 16, 16)]
                        rows_v[p, r, pl.ds(cc * 16, 16)] = v * spl
                return carry2

            lax.fori_loop(0, BE // 16, grp_body, 0, unroll=5)

        def start_scatters(p):
            sc_acc(p).start(add=True)

            @pl.when(c == 0)
            def _():
                sc_den(p).start(add=True)

        def wait_scatters(p):
            sc_acc(p).wait()

            @pl.when(c == 0)
            def _():
                sc_den(p).wait()

        start_idx(0, 0)
        wait_idx(0, 0)
        scalar_phase(0)
        gather(0).start()
        start_idx(1, 1)

        def batch_body(b, carry):
            p = lax.bitwise_and(b, 1)
            q = 1 - p
            wait_idx(b, p)

            @pl.when(b >= 2)
            def _():
                wait_scatters(p)

            scalar_phase(p)
            gather(p).start()

            @pl.when(b < NBATCH - 1)
            def _():
                start_idx(b + 1, q)

            gather(q).wait()
            multiply(q)
            start_scatters(q)
            return carry

        lax.fori_loop(1, NBATCH, batch_body, 0)
        p_last = (NBATCH - 1) & 1
        gather(p_last).wait()
        multiply(p_last)
        start_scatters(p_last)
        wait_scatters(1 - p_last)
        wait_scatters(p_last)
        plsc.subcore_barrier()
        out0 = pl.multiple_of(c * NPAD + row0, 8)
        pltpu.sync_copy(acc_sh.at[pl.ds(row0, RPT)],
                        acc_out.at[pl.ds(out0, RPT)])

        @pl.when(c == 0)
        def _out_den():
            pltpu.sync_copy(den_sh.at[pl.ds(row0, RPT)],
                            den_out.at[pl.ds(row0, RPT)])

        @pl.when(s == NS - 1)
        def _out_tail():
            t0 = pl.multiple_of(c * NPAD + NS * RPT, 8)
            pltpu.sync_copy(acc_sh.at[pl.ds(NS * RPT, NTAIL)],
                            acc_out.at[pl.ds(t0, NTAIL)])

            @pl.when(c == 0)
            def _out_den_tail():
                pltpu.sync_copy(den_sh.at[pl.ds(NS * RPT, NTAIL)],
                                den_out.at[pl.ds(NS * RPT, NTAIL)])

    return k(hflat, aux, ei, ew, wa, z64, z16)


def _dot_t(a, b):
    return lax.dot_general(a, b, (((1,), (1,)), ((), ())),
                           preferred_element_type=jnp.float32)


def _tc_embed(x, W, asv, adv):
    def body(x_ref, w_ref, as_ref, ad_ref, aux_ref, hs_ref):
        h = _dot_t(x_ref[...], w_ref[...])
        aux_ref[...] = jnp.zeros((8, BN), jnp.float32)
        aux_ref[0:1, :] = _dot_t(as_ref[...], h)
        aux_ref[1:2, :] = _dot_t(ad_ref[...], h)
        hs_ref[0] = h[:, :DH]
        hs_ref[1] = h[:, DH:]

    return pl.pallas_call(
        body,
        grid=(NBLK,),
        in_specs=[
            pl.BlockSpec((BN, D), lambda i: (i, 0)),
            pl.BlockSpec((D, D), lambda i: (0, 0)),
            pl.BlockSpec((1, D), lambda i: (0, 0)),
            pl.BlockSpec((1, D), lambda i: (0, 0)),
        ],
        out_specs=[
            pl.BlockSpec((8, BN), lambda i: (0, i)),
            pl.BlockSpec((NC, BN, DH), lambda i: (0, i, 0)),
        ],
        out_shape=[
            jax.ShapeDtypeStruct((8, N), jnp.float32),
            jax.ShapeDtypeStruct((NC, NPAD, DH), jnp.float32),
        ],
    )(x, W, asv, adv)


def _self_alpha(aux_ref, wa_ref, la):
    auxT = lax.dot_general(aux_ref[...], jnp.eye(8, dtype=jnp.float32),
                           (((0,), (0,)), ((), ())),
                           preferred_element_type=jnp.float32)
    wa = wa_ref[...]
    ce = jnp.sum(wa[0:1, :] * wa[1:2, :])
    logit = auxT[:, 0:1] + auxT[:, 1:2] + ce * la
    logit = jnp.where(logit > 0, logit, logit * jnp.float32(0.2))
    return jnp.exp(logit)


def _tc_combine1(acc, den, h1s, aux1, wa1, b1, W2, as2v, ad2v):
    def body(alo_ref, ahi_ref, den_ref, hlo_ref, hhi_ref, aux_ref, wa_ref,
             b_ref, w2_ref, as_ref, ad_ref, aux2_ref, la_ref, h2s_ref):
        dsum = den_ref[...]
        asum = dsum[:, 0:1]
        deg = dsum[:, 1:2]
        ews = dsum[:, 2:3]
        la = ews / jnp.maximum(deg, 1.0)
        aself = _self_alpha(aux_ref, wa_ref, la)
        dentot = asum + aself + jnp.float32(1e-16)
        rden = jnp.float32(1.0) / dentot
        x2lo = jnp.maximum(
            (alo_ref[0] + aself * hlo_ref[0]) * rden + b_ref[:, :DH], 0.0)
        x2hi = jnp.maximum(
            (ahi_ref[0] + aself * hhi_ref[0]) * rden + b_ref[:, DH:], 0.0)
        h2 = _dot_t(x2lo, w2_ref[...][:, :DH]) + _dot_t(x2hi, w2_ref[...][:, DH:])
        aux2_ref[...] = jnp.zeros((8, BN), jnp.float32)
        aux2_ref[0:1, :] = _dot_t(as_ref[...], h2)
        aux2_ref[1:2, :] = _dot_t(ad_ref[...], h2)
        la_ref[...] = la
        h2s_ref[0] = h2[:, :DH]
        h2s_ref[1] = h2[:, DH:]

    return pl.pallas_call(
        body,
        grid=(NBLK,),
        in_specs=[
            pl.BlockSpec((1, BN, DH), lambda i: (0, i, 0)),
            pl.BlockSpec((1, BN, DH), lambda i: (1, i, 0)),
            pl.BlockSpec((BN, 16), lambda i: (i, 0)),
            pl.BlockSpec((1, BN, DH), lambda i: (0, i, 0)),
            pl.BlockSpec((1, BN, DH), lambda i: (1, i, 0)),
            pl.BlockSpec((8, BN), lambda i: (0, i)),
            pl.BlockSpec((2, D), lambda i: (0, 0)),
            pl.BlockSpec((1, D), lambda i: (0, 0)),
            pl.BlockSpec((D, D), lambda i: (0, 0)),
            pl.BlockSpec((1, D), lambda i: (0, 0)),
            pl.BlockSpec((1, D), lambda i: (0, 0)),
        ],
        out_specs=[
            pl.BlockSpec((8, BN), lambda i: (0, i)),
            pl.BlockSpec((BN, 1), lambda i: (i, 0)),
            pl.BlockSpec((NC, BN, DH), lambda i: (0, i, 0)),
        ],
        out_shape=[
            jax.ShapeDtypeStruct((8, N), jnp.float32),
            jax.ShapeDtypeStruct((N, 1), jnp.float32),
            jax.ShapeDtypeStruct((NC, NPAD, DH), jnp.float32),
        ],
    )(acc, acc, den, h1s, h1s, aux1, wa1, b1, W2, as2v, ad2v)


def _tc_final(acc, den, h2s, aux2, la, wa2, b2):
    def body(alo_ref, ahi_ref, den_ref, hlo_ref, hhi_ref, aux_ref, la_ref,
             wa_ref, b_ref, out_ref):
        asum = den_ref[:, 0:1]
        la = la_ref[...]
        aself = _self_alpha(aux_ref, wa_ref, la)
        dentot = asum + aself + jnp.float32(1e-16)
        rden = jnp.float32(1.0) / dentot
        out_ref[:, :DH] = (alo_ref[0] + aself * hlo_ref[0]) * rden + b_ref[:, :DH]
        out_ref[:, DH:] = (ahi_ref[0] + aself * hhi_ref[0]) * rden + b_ref[:, DH:]

    return pl.pallas_call(
        body,
        grid=(NBLK,),
        in_specs=[
            pl.BlockSpec((1, BN, DH), lambda i: (0, i, 0)),
            pl.BlockSpec((1, BN, DH), lambda i: (1, i, 0)),
            pl.BlockSpec((BN, 16), lambda i: (i, 0)),
            pl.BlockSpec((1, BN, DH), lambda i: (0, i, 0)),
            pl.BlockSpec((1, BN, DH), lambda i: (1, i, 0)),
            pl.BlockSpec((8, BN), lambda i: (0, i)),
            pl.BlockSpec((BN, 1), lambda i: (i, 0)),
            pl.BlockSpec((2, D), lambda i: (0, 0)),
            pl.BlockSpec((1, D), lambda i: (0, 0)),
        ],
        out_specs=pl.BlockSpec((BN, D), lambda i: (i, 0)),
        out_shape=jax.ShapeDtypeStruct((N, D), jnp.float32),
    )(acc, acc, den, h2s, h2s, aux2, la, wa2, b2)


def kernel(x, edge_index, edge_weight, W1, We1, as1, ad1, ae1, b1,
           W2, We2, as2, ad2, ae2, b2):
    ew = edge_weight.reshape(E)
    as1v = as1.reshape(1, D)
    ad1v = ad1.reshape(1, D)
    wa1 = jnp.concatenate([We1.reshape(1, D), ae1.reshape(1, D)], axis=0)
    as2v = as2.reshape(1, D)
    ad2v = ad2.reshape(1, D)
    wa2 = jnp.concatenate([We2.reshape(1, D), ae2.reshape(1, D)], axis=0)
    b1r = b1.reshape(1, D)
    b2r = b2.reshape(1, D)
    z64 = jnp.zeros((RPT, DH), jnp.float32)
    z16 = jnp.zeros((RPT, 16), jnp.float32)

    aux1, h1s = _tc_embed(x, W1, as1v, ad1v)
    acc1, den1 = _sc_edge_pass(h1s.reshape(NC * NPAD, DH), aux1, edge_index,
                               ew, wa1, z64, z16)
    aux2, la, h2s = _tc_combine1(acc1.reshape(NC, NPAD, DH), den1, h1s,
                                 aux1, wa1, b1r, W2, as2v, ad2v)
    acc2, den2 = _sc_edge_pass(h2s.reshape(NC * NPAD, DH), aux2, edge_index,
                               ew, wa2, z64, z16)
    return _tc_final(acc2.reshape(NC, NPAD, DH), den2, h2s, aux2, la, wa2, b2r)
